# Initial kernel scaffold; baseline (speedup 1.0000x reference)
import jax
import jax.numpy as jnp
from jax.experimental import pallas as pl


def kernel(adj, node_feat, edge_feat, params):
    raise NotImplementedError("write your pallas kernel here")


# speedup vs baseline: 18.8867x; 18.8867x over previous
import functools

import jax
import jax.numpy as jnp
from jax import lax
from jax.experimental import pallas as pl
from jax.experimental.pallas import tpu as pltpu
from jax.experimental.pallas import tpu_sc as plsc

NN = 10000
EE = 320000
DH = 128
NC = 2
NS = 16
NW = NC * NS
CHUNK = 80
NCHT = (EE // NW) // CHUNK
ROWS_PT = 624
F32 = jnp.float32

_SC_PARAMS = pltpu.CompilerParams(use_tc_tiling_on_sc=False,
                                  needs_layout_passes=False)


def _dot(a, b):
    return jnp.dot(a, b, preferred_element_type=F32)


def _node_mlp_body(x, w1, b1, w2, b2, o):
    h = jnp.tanh(_dot(x[...], w1[...]) + b1[...])
    o[...] = jnp.tanh(_dot(h, w2[...]) + b2[...])


def _node_mlp(x, w1, b1, w2, b2):
    B = 1000
    return pl.pallas_call(
        _node_mlp_body,
        grid=(NN // B,),
        in_specs=[
            pl.BlockSpec((B, DH), lambda i: (i, 0)),
            pl.BlockSpec((DH, 64), lambda i: (0, 0)),
            pl.BlockSpec((1, 64), lambda i: (0, 0)),
            pl.BlockSpec((64, DH), lambda i: (0, 0)),
            pl.BlockSpec((1, DH), lambda i: (0, 0)),
        ],
        out_specs=pl.BlockSpec((B, DH), lambda i: (i, 0)),
        out_shape=jax.ShapeDtypeStruct((NN, DH), F32),
    )(x, w1, b1, w2, b2)


def _edge_mlp_body(x, w1, b1, w2, b2, we0, ae0, we1, ae1, we2, ae2, eh_o, ax_o):
    h = jnp.tanh(_dot(x[...], w1[...]) + b1[...])
    eh = jnp.tanh(_dot(h, w2[...]) + b2[...])
    eh_o[...] = eh
    c0 = _dot(eh, _dot(we0[...], ae0[...]))
    c1 = _dot(eh, _dot(we1[...], ae1[...]))
    c2 = _dot(eh, _dot(we2[...], ae2[...]))
    z = jnp.zeros((eh.shape[0], 5), F32)
    ax_o[...] = jnp.concatenate([c0, c1, c2, z], axis=1)


def _edge_mlp(x, w1, b1, w2, b2, wes, ates):
    B = 4000
    wspec = pl.BlockSpec((DH, DH), lambda i: (0, 0))
    aspec = pl.BlockSpec((DH, 1), lambda i: (0, 0))
    return pl.pallas_call(
        _edge_mlp_body,
        grid=(EE // B,),
        in_specs=[
            pl.BlockSpec((B, 16), lambda i: (i, 0)),
            pl.BlockSpec((16, 64), lambda i: (0, 0)),
            pl.BlockSpec((1, 64), lambda i: (0, 0)),
            pl.BlockSpec((64, DH), lambda i: (0, 0)),
            pl.BlockSpec((1, DH), lambda i: (0, 0)),
            wspec, aspec, wspec, aspec, wspec, aspec,
        ],
        out_specs=[
            pl.BlockSpec((B, DH), lambda i: (i, 0)),
            pl.BlockSpec((B, 8), lambda i: (i, 0)),
        ],
        out_shape=[
            jax.ShapeDtypeStruct((EE, DH), F32),
            jax.ShapeDtypeStruct((EE, 8), F32),
        ],
    )(x, w1, b1, w2, b2, wes[0], ates[0], wes[1], ates[1], wes[2], ates[2])


def _head(hidden, w, asw, adw, we, ate, seh0, seh1, cnt2):
    h = _dot(hidden, w)
    asrc = _dot(h, asw)
    adst = _dot(h, adw)
    cnt = jnp.maximum(cnt2[:, 0:1] + cnt2[:, 1:2], 1.0)
    smean = (seh0 + seh1) / cnt
    aeloop = _dot(smean, _dot(we, ate))
    al = asrc + adst + aeloop
    al = jnp.maximum(al, 0.2 * al)
    ploop = jnp.exp(al)
    nb = h.shape[0]
    aux = jnp.concatenate([asrc, adst, ploop, jnp.zeros((nb, 5), F32)], axis=1)
    return h, aux


def _combine(acc_a, acc_b, den2, h_prev, ploop_prev, bias_prev):
    num = acc_a + acc_b + ploop_prev * h_prev
    den = den2[:, 0:1] + den2[:, 1:2] + ploop_prev
    return num / (den + 1e-16) + bias_prev


def _stage0_body(nh, w, asw, adw, we, ate, seh0, seh1, cnt2, h_o, aux_o):
    h, aux = _head(nh[...], w[...], asw[...], adw[...], we[...], ate[...],
                   seh0[...], seh1[...], cnt2[...])
    h_o[...] = h
    aux_o[...] = aux


def _stage_body(acc_a, acc_b, den2, h_p, aux_p, bias_p,
                w, asw, adw, we, ate, seh0, seh1, cnt2,
                hid_o, h_o, aux_o):
    hidden = _combine(acc_a[...], acc_b[...], den2[...], h_p[...],
                      aux_p[:, 2:3], bias_p[...])
    hid_o[...] = hidden
    h, aux = _head(hidden, w[...], asw[...], adw[...], we[...], ate[...],
                   seh0[...], seh1[...], cnt2[...])
    h_o[...] = h
    aux_o[...] = aux


_B = 1000
_bspec = {
    'n': pl.BlockSpec((_B, DH), lambda i: (i, 0)),
    'x': pl.BlockSpec((_B, 8), lambda i: (i, 0)),
    'c': pl.BlockSpec((_B, 2), lambda i: (i, 0)),
    'w': pl.BlockSpec((DH, DH), lambda i: (0, 0)),
    'v': pl.BlockSpec((DH, 1), lambda i: (0, 0)),
    'b': pl.BlockSpec((1, DH), lambda i: (0, 0)),
}


def _stage0(nh, w, asw, adw, we, ate, seh0, seh1, cnt2):
    s = _bspec
    return pl.pallas_call(
        _stage0_body,
        grid=(NN // _B,),
        in_specs=[s['n'], s['w'], s['v'], s['v'], s['w'], s['v'],
                  s['n'], s['n'], s['c']],
        out_specs=[s['n'], s['x']],
        out_shape=[jax.ShapeDtypeStruct((NN, DH), F32),
                   jax.ShapeDtypeStruct((NN, 8), F32)],
    )(nh, w, asw, adw, we, ate, seh0, seh1, cnt2)


def _stage(acc_a, acc_b, den2, h_p, aux_p, bias_p, w, asw, adw, we, ate,
           seh0, seh1, cnt2):
    s = _bspec
    return pl.pallas_call(
        _stage_body,
        grid=(NN // _B,),
        in_specs=[s['n'], s['n'], s['c'], s['n'], s['x'], s['b'],
                  s['w'], s['v'], s['v'], s['w'], s['v'],
                  s['n'], s['n'], s['c']],
        out_specs=[s['n'], s['n'], s['x']],
        out_shape=[jax.ShapeDtypeStruct((NN, DH), F32),
                   jax.ShapeDtypeStruct((NN, DH), F32),
                   jax.ShapeDtypeStruct((NN, 8), F32)],
    )(acc_a, acc_b, den2, h_p, aux_p, bias_p, w, asw, adw, we, ate,
      seh0, seh1, cnt2)


def _latent_body(acc_a, acc_b, den2, h_p, aux_p, bias_p,
                 nh, h1, h2, wa, wb, wc, wd, b1, w2, b2, z_o):
    h3 = _combine(acc_a[...], acc_b[...], den2[...], h_p[...],
                  aux_p[:, 2:3], bias_p[...])
    t = (_dot(nh[...], wa[...]) + _dot(h1[...], wb[...])
         + _dot(h2[...], wc[...]) + _dot(h3, wd[...]) + b1[...])
    t = jnp.tanh(t)
    z_o[...] = jnp.tanh(_dot(t, w2[...]) + b2[...])


def _latent(acc_a, acc_b, den2, h_p, aux_p, bias_p, nh, h1, h2,
            wa, wb, wc, wd, b1, w2, b2):
    s = _bspec
    return pl.pallas_call(
        _latent_body,
        grid=(NN // _B,),
        in_specs=[s['n'], s['n'], s['c'], s['n'], s['x'], s['b'],
                  s['n'], s['n'], s['n'],
                  s['w'], s['w'], s['w'], s['w'], s['b'], s['w'], s['b']],
        out_specs=s['n'],
        out_shape=jax.ShapeDtypeStruct((NN, DH), F32),
    )(acc_a, acc_b, den2, h_p, aux_p, bias_p, nh, h1, h2,
      wa, wb, wc, wd, b1, w2, b2)


_MESH = plsc.VectorSubcoreMesh(core_axis_name="c", subcore_axis_name="s",
                               num_cores=NC, num_subcores=NS)


def _sc_prep_body(dst_r, eh_r, sehp_r, cntp_r,
                  dst_t, rows_t, ones_t, cz_t, seh_s, cnt_s):
    cid = lax.axis_index("c")
    sid = lax.axis_index("s")
    tile = cid * NS + sid

    def zb(i, c):
        r = i // 8
        col = (i % 8) * 16
        rows_t[r, pl.ds(col, 16)] = jnp.zeros((16,), F32)
        return c
    lax.fori_loop(0, CHUNK * 8, zb, 0)

    def zc(i, c):
        cz_t[pl.ds(i * 16, 16)] = jnp.zeros((16,), F32)
        return c
    lax.fori_loop(0, 39, zc, 0)

    def ob(i, c):
        ones_t[pl.ds(i * 16, 16)] = jnp.full((16,), 1.0, F32)
        return c
    lax.fori_loop(0, 5, ob, 0)

    base = sid * ROWS_PT
    for q in range(7):
        pltpu.sync_copy(rows_t, seh_s.at[pl.ds(base + q * CHUNK, CHUNK)])
    pltpu.sync_copy(rows_t.at[pl.ds(0, 64)], seh_s.at[pl.ds(base + 560, 64)])
    pltpu.sync_copy(cz_t, cnt_s.at[pl.ds(base, ROWS_PT)])

    @pl.when(sid == NS - 1)
    def _():
        pltpu.sync_copy(rows_t.at[pl.ds(0, 16)], seh_s.at[pl.ds(9984, 16)])
        pltpu.sync_copy(cz_t.at[pl.ds(0, 16)], cnt_s.at[pl.ds(9984, 16)])

    pltpu.sync_copy(dst_r.at[pl.ds(tile * NCHT, NCHT)], dst_t)
    plsc.subcore_barrier()

    ebase = tile * (CHUNK * NCHT)

    def cb(j, c):
        pltpu.sync_copy(eh_r.at[pl.ds(ebase + j * CHUNK, CHUNK)], rows_t)
        pltpu.sync_copy(rows_t, seh_s.at[dst_t.at[j]], add=True)
        pltpu.sync_copy(ones_t, cnt_s.at[dst_t.at[j]], add=True)
        return c
    lax.fori_loop(0, NCHT, cb, 0)

    plsc.subcore_barrier()
    pltpu.sync_copy(seh_s.at[pl.ds(base, ROWS_PT)],
                    sehp_r.at[cid, pl.ds(base, ROWS_PT)])
    pltpu.sync_copy(cnt_s.at[pl.ds(base, ROWS_PT)],
                    cntp_r.at[cid, pl.ds(base, ROWS_PT)])

    @pl.when(sid == NS - 1)
    def _():
        pltpu.sync_copy(seh_s.at[pl.ds(9984, 16)],
                        sehp_r.at[cid, pl.ds(9984, 16)])
        pltpu.sync_copy(cnt_s.at[pl.ds(9984, 16)],
                        cntp_r.at[cid, pl.ds(9984, 16)])


def _sc_prep(dst2d, eh):
    return pl.kernel(
        _sc_prep_body,
        out_type=[jax.ShapeDtypeStruct((NC, NN, DH), F32),
                  jax.ShapeDtypeStruct((NC, NN), F32)],
        mesh=_MESH,
        compiler_params=_SC_PARAMS,
        scratch_types=[
            pltpu.VMEM((NCHT, CHUNK), jnp.int32),
            pltpu.VMEM((CHUNK, DH), F32),
            pltpu.VMEM((CHUNK,), F32),
            pltpu.VMEM((ROWS_PT,), F32),
            pltpu.VMEM_SHARED((NN, DH), F32),
            pltpu.VMEM_SHARED((NN,), F32),
        ],
    )(dst2d, eh)


def _sc_pass_body(src_r, dst_r, ae_r, asrc_r, adst_r, h_r, accp_r, denp_r,
                  src_t, dst_t, ae_t, asv_t, adv_t, p_t, cz_t, rows_t,
                  acc_s, den_s, asrc_s, adst_s):
    cid = lax.axis_index("c")
    sid = lax.axis_index("s")
    tile = cid * NS + sid

    def zb(i, c):
        r = i // 8
        col = (i % 8) * 16
        rows_t[r, pl.ds(col, 16)] = jnp.zeros((16,), F32)
        return c
    lax.fori_loop(0, CHUNK * 8, zb, 0)

    base = sid * ROWS_PT

    pltpu.sync_copy(asrc_r.at[pl.ds(base, ROWS_PT)], cz_t)
    pltpu.sync_copy(cz_t, asrc_s.at[pl.ds(base, ROWS_PT)])
    pltpu.sync_copy(adst_r.at[pl.ds(base, ROWS_PT)], cz_t)
    pltpu.sync_copy(cz_t, adst_s.at[pl.ds(base, ROWS_PT)])

    @pl.when(sid == NS - 1)
    def _():
        pltpu.sync_copy(asrc_r.at[pl.ds(9984, 16)], cz_t.at[pl.ds(0, 16)])
        pltpu.sync_copy(cz_t.at[pl.ds(0, 16)], asrc_s.at[pl.ds(9984, 16)])
        pltpu.sync_copy(adst_r.at[pl.ds(9984, 16)], cz_t.at[pl.ds(0, 16)])
        pltpu.sync_copy(cz_t.at[pl.ds(0, 16)], adst_s.at[pl.ds(9984, 16)])

    def zc(i, c):
        cz_t[pl.ds(i * 16, 16)] = jnp.zeros((16,), F32)
        return c
    lax.fori_loop(0, 39, zc, 0)
    for q in range(7):
        pltpu.sync_copy(rows_t, acc_s.at[pl.ds(base + q * CHUNK, CHUNK)])
    pltpu.sync_copy(rows_t.at[pl.ds(0, 64)], acc_s.at[pl.ds(base + 560, 64)])
    pltpu.sync_copy(cz_t, den_s.at[pl.ds(base, ROWS_PT)])

    @pl.when(sid == NS - 1)
    def _():
        pltpu.sync_copy(rows_t.at[pl.ds(0, 16)], acc_s.at[pl.ds(9984, 16)])
        pltpu.sync_copy(cz_t.at[pl.ds(0, 16)], den_s.at[pl.ds(9984, 16)])

    pltpu.sync_copy(src_r.at[pl.ds(tile * NCHT, NCHT)], src_t)
    pltpu.sync_copy(dst_r.at[pl.ds(tile * NCHT, NCHT)], dst_t)
    pltpu.sync_copy(ae_r.at[pl.ds(tile * NCHT, NCHT)], ae_t)
    plsc.subcore_barrier()

    def cb(j, c):
        pltpu.sync_copy(asrc_s.at[src_t.at[j]], asv_t)
        pltpu.sync_copy(adst_s.at[dst_t.at[j]], adv_t)
        for k in range(CHUNK // 16):
            a = (asv_t[pl.ds(k * 16, 16)]
                 + adv_t[pl.ds(k * 16, 16)]
                 + ae_t[j, pl.ds(k * 16, 16)])
            a = jnp.maximum(a, 0.2 * a)
            p_t[pl.ds(k * 16, 16)] = jnp.exp(a)
        pltpu.sync_copy(h_r.at[src_t.at[j]], rows_t)

        def rb(r, c2):
            pv = plsc.load_gather(p_t, [jnp.full((16,), r, jnp.int32)])
            for cc in range(8):
                rows_t[r, pl.ds(cc * 16, 16)] = rows_t[r, pl.ds(cc * 16, 16)] * pv
            return c2
        lax.fori_loop(0, CHUNK, rb, 0)
        pltpu.sync_copy(rows_t, acc_s.at[dst_t.at[j]], add=True)
        pltpu.sync_copy(p_t, den_s.at[dst_t.at[j]], add=True)
        return c
    lax.fori_loop(0, NCHT, cb, 0)

    plsc.subcore_barrier()
    pltpu.sync_copy(acc_s.at[pl.ds(base, ROWS_PT)],
                    accp_r.at[cid, pl.ds(base, ROWS_PT)])
    pltpu.sync_copy(den_s.at[pl.ds(base, ROWS_PT)],
                    denp_r.at[cid, pl.ds(base, ROWS_PT)])

    @pl.when(sid == NS - 1)
    def _():
        pltpu.sync_copy(acc_s.at[pl.ds(9984, 16)],
                        accp_r.at[cid, pl.ds(9984, 16)])
        pltpu.sync_copy(den_s.at[pl.ds(9984, 16)],
                        denp_r.at[cid, pl.ds(9984, 16)])


def _sc_pass(src2d, dst2d, ae2d, asrc, adst, h):
    return pl.kernel(
        _sc_pass_body,
        out_type=[jax.ShapeDtypeStruct((NC, NN, DH), F32),
                  jax.ShapeDtypeStruct((NC, NN), F32)],
        mesh=_MESH,
        compiler_params=_SC_PARAMS,
        scratch_types=[
            pltpu.VMEM((NCHT, CHUNK), jnp.int32),
            pltpu.VMEM((NCHT, CHUNK), jnp.int32),
            pltpu.VMEM((NCHT, CHUNK), F32),
            pltpu.VMEM((CHUNK,), F32),
            pltpu.VMEM((CHUNK,), F32),
            pltpu.VMEM((CHUNK,), F32),
            pltpu.VMEM((ROWS_PT,), F32),
            pltpu.VMEM((CHUNK, DH), F32),
            pltpu.VMEM_SHARED((NN, DH), F32),
            pltpu.VMEM_SHARED((NN,), F32),
            pltpu.VMEM_SHARED((NN,), F32),
            pltpu.VMEM_SHARED((NN,), F32),
        ],
    )(src2d, dst2d, ae2d, asrc, adst, h)


def kernel(adj, node_feat, edge_feat, params):
    p = params
    src2d = adj[0].reshape(EE // CHUNK, CHUNK)
    dst2d = adj[1].reshape(EE // CHUNK, CHUNK)

    nh = _node_mlp(node_feat,
                   p['node_fc1_W'], p['node_fc1_b'].reshape(1, 64),
                   p['node_fc2_W'], p['node_fc2_b'].reshape(1, DH))

    wes = [p['gat%d_W_edge' % l] for l in range(3)]
    ates = [p['gat%d_att_edge' % l].reshape(DH, 1) for l in range(3)]
    eh, aex = _edge_mlp(edge_feat,
                        p['edge_fc1_W'], p['edge_fc1_b'].reshape(1, 64),
                        p['edge_fc2_W'], p['edge_fc2_b'].reshape(1, DH),
                        wes, ates)

    sehp, cntp = _sc_prep(dst2d, eh)
    seh0, seh1 = sehp[0], sehp[1]
    cnt2 = cntp.T

    def layer_w(l):
        return (p['gat%d_W' % l],
                p['gat%d_att_src' % l].reshape(DH, 1),
                p['gat%d_att_dst' % l].reshape(DH, 1),
                p['gat%d_W_edge' % l],
                p['gat%d_att_edge' % l].reshape(DH, 1))

    hcur, aux = _stage0(nh, *layer_w(0), seh0, seh1, cnt2)
    hs = [nh]
    z = None
    for l in range(3):
        ae2d = aex[:, l].reshape(EE // CHUNK, CHUNK)
        accp, denp = _sc_pass(src2d, dst2d, ae2d, aux[:, 0], aux[:, 1], hcur)
        den2 = denp.T
        bias_p = p['gat%d_bias' % l].reshape(1, DH)
        if l < 2:
            hidden, hcur, aux = _stage(accp[0], accp[1], den2, hcur, aux,
                                       bias_p, *layer_w(l + 1),
                                       seh0, seh1, cnt2)
            hs.append(hidden)
        else:
            lw = p['latent_fc1_W']
            z = _latent(accp[0], accp[1], den2, hcur, aux, bias_p,
                        hs[0], hs[1], hs[2],
                        lw[0:DH], lw[DH:2 * DH], lw[2 * DH:3 * DH],
                        lw[3 * DH:4 * DH],
                        p['latent_fc1_b'].reshape(1, DH),
                        p['latent_fc2_W'],
                        p['latent_fc2_b'].reshape(1, DH))
    return z, eh


# speedup vs baseline: 26.8921x; 1.4239x over previous
import functools

import jax
import jax.numpy as jnp
from jax import lax
from jax.experimental import pallas as pl
from jax.experimental.pallas import tpu as pltpu
from jax.experimental.pallas import tpu_sc as plsc

NN = 10000
EE = 320000
DH = 128
NC = 2
NS = 16
NW = NC * NS
CHUNK = 80
NCHT = (EE // NW) // CHUNK
ROWS_PT = 624
F32 = jnp.float32

_SC_PARAMS = pltpu.CompilerParams(use_tc_tiling_on_sc=False,
                                  needs_layout_passes=False)


def _dot(a, b):
    return jnp.dot(a, b, preferred_element_type=F32)


def _node_mlp_body(x, w1, b1, w2, b2, o):
    h = jnp.tanh(_dot(x[...], w1[...]) + b1[...])
    o[...] = jnp.tanh(_dot(h, w2[...]) + b2[...])


def _node_mlp(x, w1, b1, w2, b2):
    B = 1000
    return pl.pallas_call(
        _node_mlp_body,
        grid=(NN // B,),
        in_specs=[
            pl.BlockSpec((B, DH), lambda i: (i, 0)),
            pl.BlockSpec((DH, 64), lambda i: (0, 0)),
            pl.BlockSpec((1, 64), lambda i: (0, 0)),
            pl.BlockSpec((64, DH), lambda i: (0, 0)),
            pl.BlockSpec((1, DH), lambda i: (0, 0)),
        ],
        out_specs=pl.BlockSpec((B, DH), lambda i: (i, 0)),
        out_shape=jax.ShapeDtypeStruct((NN, DH), F32),
    )(x, w1, b1, w2, b2)


def _edge_mlp_body(x, w1, b1, w2, b2, we0, ae0, we1, ae1, we2, ae2, eh_o, ax_o):
    h = jnp.tanh(_dot(x[...], w1[...]) + b1[...])
    eh = jnp.tanh(_dot(h, w2[...]) + b2[...])
    eh_o[...] = eh
    c0 = _dot(eh, _dot(we0[...], ae0[...]))
    c1 = _dot(eh, _dot(we1[...], ae1[...]))
    c2 = _dot(eh, _dot(we2[...], ae2[...]))
    z = jnp.zeros((eh.shape[0], 5), F32)
    ax_o[...] = jnp.concatenate([c0, c1, c2, z], axis=1)


def _edge_mlp(x, w1, b1, w2, b2, wes, ates):
    B = 4000
    wspec = pl.BlockSpec((DH, DH), lambda i: (0, 0))
    aspec = pl.BlockSpec((DH, 1), lambda i: (0, 0))
    return pl.pallas_call(
        _edge_mlp_body,
        grid=(EE // B,),
        in_specs=[
            pl.BlockSpec((B, 16), lambda i: (i, 0)),
            pl.BlockSpec((16, 64), lambda i: (0, 0)),
            pl.BlockSpec((1, 64), lambda i: (0, 0)),
            pl.BlockSpec((64, DH), lambda i: (0, 0)),
            pl.BlockSpec((1, DH), lambda i: (0, 0)),
            wspec, aspec, wspec, aspec, wspec, aspec,
        ],
        out_specs=[
            pl.BlockSpec((B, DH), lambda i: (i, 0)),
            pl.BlockSpec((B, 8), lambda i: (i, 0)),
        ],
        out_shape=[
            jax.ShapeDtypeStruct((EE, DH), F32),
            jax.ShapeDtypeStruct((EE, 8), F32),
        ],
    )(x, w1, b1, w2, b2, wes[0], ates[0], wes[1], ates[1], wes[2], ates[2])


def _head(hidden, w, asw, adw, we, ate, seh0, seh1, cnt2):
    h = _dot(hidden, w)
    asrc = _dot(h, asw)
    adst = _dot(h, adw)
    cnt = jnp.maximum(cnt2[:, 0:1] + cnt2[:, 1:2], 1.0)
    smean = (seh0 + seh1) / cnt
    aeloop = _dot(smean, _dot(we, ate))
    al = asrc + adst + aeloop
    al = jnp.maximum(al, 0.2 * al)
    ploop = jnp.exp(al)
    nb = h.shape[0]
    aux = jnp.concatenate([asrc, adst, ploop, jnp.zeros((nb, 5), F32)], axis=1)
    return h, aux


def _combine(acc_a, acc_b, den2, h_prev, ploop_prev, bias_prev):
    num = acc_a + acc_b + ploop_prev * h_prev
    den = den2[:, 0:1] + den2[:, 1:2] + ploop_prev
    return num / (den + 1e-16) + bias_prev


def _stage0_body(nh, w, asw, adw, we, ate, seh0, seh1, cnt2, h_o, aux_o):
    h, aux = _head(nh[...], w[...], asw[...], adw[...], we[...], ate[...],
                   seh0[...], seh1[...], cnt2[...])
    h_o[...] = h
    aux_o[...] = aux


def _stage_body(acc_a, acc_b, den2, h_p, aux_p, bias_p,
                w, asw, adw, we, ate, seh0, seh1, cnt2,
                hid_o, h_o, aux_o):
    hidden = _combine(acc_a[...], acc_b[...], den2[...], h_p[...],
                      aux_p[:, 2:3], bias_p[...])
    hid_o[...] = hidden
    h, aux = _head(hidden, w[...], asw[...], adw[...], we[...], ate[...],
                   seh0[...], seh1[...], cnt2[...])
    h_o[...] = h
    aux_o[...] = aux


_B = 1000
_bspec = {
    'n': pl.BlockSpec((_B, DH), lambda i: (i, 0)),
    'x': pl.BlockSpec((_B, 8), lambda i: (i, 0)),
    'c': pl.BlockSpec((_B, 2), lambda i: (i, 0)),
    'w': pl.BlockSpec((DH, DH), lambda i: (0, 0)),
    'v': pl.BlockSpec((DH, 1), lambda i: (0, 0)),
    'b': pl.BlockSpec((1, DH), lambda i: (0, 0)),
}


def _stage0(nh, w, asw, adw, we, ate, seh0, seh1, cnt2):
    s = _bspec
    return pl.pallas_call(
        _stage0_body,
        grid=(NN // _B,),
        in_specs=[s['n'], s['w'], s['v'], s['v'], s['w'], s['v'],
                  s['n'], s['n'], s['c']],
        out_specs=[s['n'], s['x']],
        out_shape=[jax.ShapeDtypeStruct((NN, DH), F32),
                   jax.ShapeDtypeStruct((NN, 8), F32)],
    )(nh, w, asw, adw, we, ate, seh0, seh1, cnt2)


def _stage(acc_a, acc_b, den2, h_p, aux_p, bias_p, w, asw, adw, we, ate,
           seh0, seh1, cnt2):
    s = _bspec
    return pl.pallas_call(
        _stage_body,
        grid=(NN // _B,),
        in_specs=[s['n'], s['n'], s['c'], s['n'], s['x'], s['b'],
                  s['w'], s['v'], s['v'], s['w'], s['v'],
                  s['n'], s['n'], s['c']],
        out_specs=[s['n'], s['n'], s['x']],
        out_shape=[jax.ShapeDtypeStruct((NN, DH), F32),
                   jax.ShapeDtypeStruct((NN, DH), F32),
                   jax.ShapeDtypeStruct((NN, 8), F32)],
    )(acc_a, acc_b, den2, h_p, aux_p, bias_p, w, asw, adw, we, ate,
      seh0, seh1, cnt2)


def _latent_body(acc_a, acc_b, den2, h_p, aux_p, bias_p,
                 nh, h1, h2, wa, wb, wc, wd, b1, w2, b2, z_o):
    h3 = _combine(acc_a[...], acc_b[...], den2[...], h_p[...],
                  aux_p[:, 2:3], bias_p[...])
    t = (_dot(nh[...], wa[...]) + _dot(h1[...], wb[...])
         + _dot(h2[...], wc[...]) + _dot(h3, wd[...]) + b1[...])
    t = jnp.tanh(t)
    z_o[...] = jnp.tanh(_dot(t, w2[...]) + b2[...])


def _latent(acc_a, acc_b, den2, h_p, aux_p, bias_p, nh, h1, h2,
            wa, wb, wc, wd, b1, w2, b2):
    s = _bspec
    return pl.pallas_call(
        _latent_body,
        grid=(NN // _B,),
        in_specs=[s['n'], s['n'], s['c'], s['n'], s['x'], s['b'],
                  s['n'], s['n'], s['n'],
                  s['w'], s['w'], s['w'], s['w'], s['b'], s['w'], s['b']],
        out_specs=s['n'],
        out_shape=jax.ShapeDtypeStruct((NN, DH), F32),
    )(acc_a, acc_b, den2, h_p, aux_p, bias_p, nh, h1, h2,
      wa, wb, wc, wd, b1, w2, b2)


_MESH = plsc.VectorSubcoreMesh(core_axis_name="c", subcore_axis_name="s",
                               num_cores=NC, num_subcores=NS)


def _sc_prep_body(dst_r, eh_r, sehp_r, cntp_r,
                  dst_t, rows2_t, ones_t, cz_t, seh_s, cnt_s,
                  semg0, semg1, sems0, sems1):
    cid = lax.axis_index("c")
    sid = lax.axis_index("s")
    tile = cid * NS + sid
    rows_t = rows2_t.at[0]

    def zb(i, c):
        r = i // 8
        col = (i % 8) * 16
        rows2_t[0, r, pl.ds(col, 16)] = jnp.zeros((16,), F32)
        return c
    lax.fori_loop(0, CHUNK * 8, zb, 0)

    def zc(i, c):
        cz_t[pl.ds(i * 16, 16)] = jnp.zeros((16,), F32)
        return c
    lax.fori_loop(0, 39, zc, 0)

    def ob(i, c):
        ones_t[pl.ds(i * 16, 16)] = jnp.full((16,), 1.0, F32)
        return c
    lax.fori_loop(0, 5, ob, 0)

    base = sid * ROWS_PT
    for q in range(7):
        pltpu.sync_copy(rows_t, seh_s.at[pl.ds(base + q * CHUNK, CHUNK)])
    pltpu.sync_copy(rows_t.at[pl.ds(0, 64)], seh_s.at[pl.ds(base + 560, 64)])
    pltpu.sync_copy(cz_t, cnt_s.at[pl.ds(base, ROWS_PT)])

    @pl.when(sid == NS - 1)
    def _():
        pltpu.sync_copy(rows_t.at[pl.ds(0, 16)], seh_s.at[pl.ds(9984, 16)])
        pltpu.sync_copy(cz_t.at[pl.ds(0, 16)], cnt_s.at[pl.ds(9984, 16)])

    pltpu.sync_copy(dst_r.at[pl.ds(tile * NCHT, NCHT)], dst_t)
    plsc.subcore_barrier()

    ebase = tile * (CHUNK * NCHT)

    def g_issue(j, b, sem):
        pltpu.async_copy(eh_r.at[pl.ds(ebase + j * CHUNK, CHUNK)],
                         rows2_t.at[b], sem)

    def g_wait(j, b, sem):
        pltpu.make_async_copy(eh_r.at[pl.ds(ebase + j * CHUNK, CHUNK)],
                              rows2_t.at[b], sem).wait()

    def s_issue(j, b, sem):
        pltpu.async_copy(rows2_t.at[b], seh_s.at[dst_t.at[j]], sem, add=True)

    def s_wait(j, b, sem):
        pltpu.make_async_copy(rows2_t.at[b], seh_s.at[dst_t.at[j]], sem).wait()

    g_issue(0, 0, semg0)
    g_issue(1, 1, semg1)

    def cb(g, c):
        for b, sg, ss in ((0, semg0, sems0), (1, semg1, sems1)):
            j = g + b

            @pl.when(j < NCHT)
            def _():
                g_wait(j, b, sg)
                s_issue(j, b, ss)
                pltpu.sync_copy(ones_t, cnt_s.at[dst_t.at[j]], add=True)
                s_wait(j, b, ss)

                @pl.when(j + 2 < NCHT)
                def _():
                    g_issue(j + 2, b, sg)
        return c
    lax.fori_loop(0, (NCHT + 1) // 2, lambda i, c: cb(2 * i, c), 0)

    plsc.subcore_barrier()
    pltpu.sync_copy(seh_s.at[pl.ds(base, ROWS_PT)],
                    sehp_r.at[cid, pl.ds(base, ROWS_PT)])
    pltpu.sync_copy(cnt_s.at[pl.ds(base, ROWS_PT)],
                    cntp_r.at[cid, pl.ds(base, ROWS_PT)])

    @pl.when(sid == NS - 1)
    def _():
        pltpu.sync_copy(seh_s.at[pl.ds(9984, 16)],
                        sehp_r.at[cid, pl.ds(9984, 16)])
        pltpu.sync_copy(cnt_s.at[pl.ds(9984, 16)],
                        cntp_r.at[cid, pl.ds(9984, 16)])


def _sc_prep(dst2d, eh):
    return pl.kernel(
        _sc_prep_body,
        out_type=[jax.ShapeDtypeStruct((NC, NN, DH), F32),
                  jax.ShapeDtypeStruct((NC, NN), F32)],
        mesh=_MESH,
        compiler_params=_SC_PARAMS,
        scratch_types=[
            pltpu.VMEM((NCHT, CHUNK), jnp.int32),
            pltpu.VMEM((2, CHUNK, DH), F32),
            pltpu.VMEM((CHUNK,), F32),
            pltpu.VMEM((ROWS_PT,), F32),
            pltpu.VMEM_SHARED((NN, DH), F32),
            pltpu.VMEM_SHARED((NN,), F32),
            pltpu.SemaphoreType.DMA,
            pltpu.SemaphoreType.DMA,
            pltpu.SemaphoreType.DMA,
            pltpu.SemaphoreType.DMA,
        ],
    )(dst2d, eh)


def _sc_pass_body(src_r, dst_r, ae_r, asrc_r, adst_r, h_r, accp_r, denp_r,
                  src_t, dst_t, p_t, cz_t, rows2_t, asv2_t, adv2_t, aev2_t,
                  acc_s, den_s, semi0, semi1, sems0, sems1):
    cid = lax.axis_index("c")
    sid = lax.axis_index("s")
    tile = cid * NS + sid
    rows_t = rows2_t.at[0]

    def zb(i, c):
        r = i // 8
        col = (i % 8) * 16
        rows2_t[0, r, pl.ds(col, 16)] = jnp.zeros((16,), F32)
        return c
    lax.fori_loop(0, CHUNK * 8, zb, 0)

    def zc(i, c):
        cz_t[pl.ds(i * 16, 16)] = jnp.zeros((16,), F32)
        return c
    lax.fori_loop(0, 39, zc, 0)

    base = sid * ROWS_PT
    for q in range(7):
        pltpu.sync_copy(rows_t, acc_s.at[pl.ds(base + q * CHUNK, CHUNK)])
    pltpu.sync_copy(rows_t.at[pl.ds(0, 64)], acc_s.at[pl.ds(base + 560, 64)])
    pltpu.sync_copy(cz_t, den_s.at[pl.ds(base, ROWS_PT)])

    @pl.when(sid == NS - 1)
    def _():
        pltpu.sync_copy(rows_t.at[pl.ds(0, 16)], acc_s.at[pl.ds(9984, 16)])
        pltpu.sync_copy(cz_t.at[pl.ds(0, 16)], den_s.at[pl.ds(9984, 16)])

    pltpu.sync_copy(src_r.at[pl.ds(tile * NCHT, NCHT)], src_t)
    pltpu.sync_copy(dst_r.at[pl.ds(tile * NCHT, NCHT)], dst_t)
    plsc.subcore_barrier()

    def in_issue(j, b, sem):
        pltpu.async_copy(asrc_r.at[src_t.at[j]], asv2_t.at[b], sem)
        pltpu.async_copy(adst_r.at[dst_t.at[j]], adv2_t.at[b], sem)
        pltpu.async_copy(ae_r.at[tile * NCHT + j], aev2_t.at[b], sem)
        pltpu.async_copy(h_r.at[src_t.at[j]], rows2_t.at[b], sem)

    def in_wait(j, b, sem):
        pltpu.make_async_copy(asrc_r.at[src_t.at[j]], asv2_t.at[b], sem).wait()
        pltpu.make_async_copy(adst_r.at[dst_t.at[j]], adv2_t.at[b], sem).wait()
        pltpu.make_async_copy(ae_r.at[tile * NCHT + j], aev2_t.at[b], sem).wait()
        pltpu.make_async_copy(h_r.at[src_t.at[j]], rows2_t.at[b], sem).wait()

    in_issue(0, 0, semi0)
    in_issue(1, 1, semi1)

    def cb(g, c):
        for b, sg, ss in ((0, semi0, sems0), (1, semi1, sems1)):
            j = g + b

            @pl.when(j < NCHT)
            def _():
                in_wait(j, b, sg)
                for k in range(CHUNK // 16):
                    a = (asv2_t[b, pl.ds(k * 16, 16)]
                         + adv2_t[b, pl.ds(k * 16, 16)]
                         + aev2_t[b, pl.ds(k * 16, 16)])
                    a = jnp.maximum(a, 0.2 * a)
                    p_t[pl.ds(k * 16, 16)] = jnp.exp(a)

                def rb8(i, c2):
                    for rr in range(8):
                        r = i * 8 + rr
                        pv = plsc.load_gather(p_t, [jnp.full((16,), r, jnp.int32)])
                        for cc in range(8):
                            rows2_t[b, r, pl.ds(cc * 16, 16)] = (
                                rows2_t[b, r, pl.ds(cc * 16, 16)] * pv)
                    return c2
                lax.fori_loop(0, CHUNK // 8, rb8, 0)

                pltpu.sync_copy(p_t, den_s.at[dst_t.at[j]], add=True)
                pltpu.async_copy(rows2_t.at[b], acc_s.at[dst_t.at[j]], ss,
                                 add=True)
                pltpu.make_async_copy(rows2_t.at[b], acc_s.at[dst_t.at[j]],
                                      ss).wait()

                @pl.when(j + 2 < NCHT)
                def _():
                    in_issue(j + 2, b, sg)
        return c
    lax.fori_loop(0, (NCHT + 1) // 2, lambda i, c: cb(2 * i, c), 0)

    plsc.subcore_barrier()
    pltpu.sync_copy(acc_s.at[pl.ds(base, ROWS_PT)],
                    accp_r.at[cid, pl.ds(base, ROWS_PT)])
    pltpu.sync_copy(den_s.at[pl.ds(base, ROWS_PT)],
                    denp_r.at[cid, pl.ds(base, ROWS_PT)])

    @pl.when(sid == NS - 1)
    def _():
        pltpu.sync_copy(acc_s.at[pl.ds(9984, 16)],
                        accp_r.at[cid, pl.ds(9984, 16)])
        pltpu.sync_copy(den_s.at[pl.ds(9984, 16)],
                        denp_r.at[cid, pl.ds(9984, 16)])


def _sc_pass(src2d, dst2d, ae2d, asrc, adst, h):
    return pl.kernel(
        _sc_pass_body,
        out_type=[jax.ShapeDtypeStruct((NC, NN, DH), F32),
                  jax.ShapeDtypeStruct((NC, NN), F32)],
        mesh=_MESH,
        compiler_params=_SC_PARAMS,
        scratch_types=[
            pltpu.VMEM((NCHT, CHUNK), jnp.int32),
            pltpu.VMEM((NCHT, CHUNK), jnp.int32),
            pltpu.VMEM((CHUNK,), F32),
            pltpu.VMEM((ROWS_PT,), F32),
            pltpu.VMEM((2, CHUNK, DH), F32),
            pltpu.VMEM((2, CHUNK), F32),
            pltpu.VMEM((2, CHUNK), F32),
            pltpu.VMEM((2, CHUNK), F32),
            pltpu.VMEM_SHARED((NN, DH), F32),
            pltpu.VMEM_SHARED((NN,), F32),
            pltpu.SemaphoreType.DMA,
            pltpu.SemaphoreType.DMA,
            pltpu.SemaphoreType.DMA,
            pltpu.SemaphoreType.DMA,
        ],
    )(src2d, dst2d, ae2d, asrc, adst, h)


def kernel(adj, node_feat, edge_feat, params):
    p = params
    src2d = adj[0].reshape(EE // CHUNK, CHUNK)
    dst2d = adj[1].reshape(EE // CHUNK, CHUNK)

    nh = _node_mlp(node_feat,
                   p['node_fc1_W'], p['node_fc1_b'].reshape(1, 64),
                   p['node_fc2_W'], p['node_fc2_b'].reshape(1, DH))

    wes = [p['gat%d_W_edge' % l] for l in range(3)]
    ates = [p['gat%d_att_edge' % l].reshape(DH, 1) for l in range(3)]
    eh, aex = _edge_mlp(edge_feat,
                        p['edge_fc1_W'], p['edge_fc1_b'].reshape(1, 64),
                        p['edge_fc2_W'], p['edge_fc2_b'].reshape(1, DH),
                        wes, ates)

    sehp, cntp = _sc_prep(dst2d, eh)
    seh0, seh1 = sehp[0], sehp[1]
    cnt2 = cntp.T

    def layer_w(l):
        return (p['gat%d_W' % l],
                p['gat%d_att_src' % l].reshape(DH, 1),
                p['gat%d_att_dst' % l].reshape(DH, 1),
                p['gat%d_W_edge' % l],
                p['gat%d_att_edge' % l].reshape(DH, 1))

    hcur, aux = _stage0(nh, *layer_w(0), seh0, seh1, cnt2)
    hs = [nh]
    z = None
    for l in range(3):
        ae2d = aex[:, l].reshape(EE // CHUNK, CHUNK)
        accp, denp = _sc_pass(src2d, dst2d, ae2d, aux[:, 0], aux[:, 1], hcur)
        den2 = denp.T
        bias_p = p['gat%d_bias' % l].reshape(1, DH)
        if l < 2:
            hidden, hcur, aux = _stage(accp[0], accp[1], den2, hcur, aux,
                                       bias_p, *layer_w(l + 1),
                                       seh0, seh1, cnt2)
            hs.append(hidden)
        else:
            lw = p['latent_fc1_W']
            z = _latent(accp[0], accp[1], den2, hcur, aux, bias_p,
                        hs[0], hs[1], hs[2],
                        lw[0:DH], lw[DH:2 * DH], lw[2 * DH:3 * DH],
                        lw[3 * DH:4 * DH],
                        p['latent_fc1_b'].reshape(1, DH),
                        p['latent_fc2_W'],
                        p['latent_fc2_b'].reshape(1, DH))
    return z, eh


# speedup vs baseline: 28.2885x; 1.0519x over previous
import functools

import jax
import jax.numpy as jnp
from jax import lax
from jax.experimental import pallas as pl
from jax.experimental.pallas import tpu as pltpu
from jax.experimental.pallas import tpu_sc as plsc

NN = 10000
EE = 320000
DH = 128
NC = 2
NS = 16
NW = NC * NS
CHUNK = 80
NCHT = (EE // NW) // CHUNK
ROWS_PT = 624
F32 = jnp.float32

_SC_PARAMS = pltpu.CompilerParams(use_tc_tiling_on_sc=False,
                                  needs_layout_passes=False)


def _dot(a, b):
    return jnp.dot(a, b, preferred_element_type=F32)


def _node_mlp_body(x, w1, b1, w2, b2, o):
    h = jnp.tanh(_dot(x[...], w1[...]) + b1[...])
    o[...] = jnp.tanh(_dot(h, w2[...]) + b2[...])


def _node_mlp(x, w1, b1, w2, b2):
    B = 1000
    return pl.pallas_call(
        _node_mlp_body,
        grid=(NN // B,),
        in_specs=[
            pl.BlockSpec((B, DH), lambda i: (i, 0)),
            pl.BlockSpec((DH, 64), lambda i: (0, 0)),
            pl.BlockSpec((1, 64), lambda i: (0, 0)),
            pl.BlockSpec((64, DH), lambda i: (0, 0)),
            pl.BlockSpec((1, DH), lambda i: (0, 0)),
        ],
        out_specs=pl.BlockSpec((B, DH), lambda i: (i, 0)),
        out_shape=jax.ShapeDtypeStruct((NN, DH), F32),
    )(x, w1, b1, w2, b2)


def _edge_mlp_body(x, w1, b1, w2, b2, we0, ae0, we1, ae1, we2, ae2, eh_o, ax_o):
    h = jnp.tanh(_dot(x[...], w1[...]) + b1[...])
    eh = jnp.tanh(_dot(h, w2[...]) + b2[...])
    eh_o[...] = eh
    c0 = _dot(eh, _dot(we0[...], ae0[...]))
    c1 = _dot(eh, _dot(we1[...], ae1[...]))
    c2 = _dot(eh, _dot(we2[...], ae2[...]))
    z = jnp.zeros((eh.shape[0], 5), F32)
    ax_o[...] = jnp.concatenate([c0, c1, c2, z], axis=1)


def _edge_mlp(x, w1, b1, w2, b2, wes, ates):
    B = 4000
    wspec = pl.BlockSpec((DH, DH), lambda i: (0, 0))
    aspec = pl.BlockSpec((DH, 1), lambda i: (0, 0))
    return pl.pallas_call(
        _edge_mlp_body,
        grid=(EE // B,),
        in_specs=[
            pl.BlockSpec((B, 16), lambda i: (i, 0)),
            pl.BlockSpec((16, 64), lambda i: (0, 0)),
            pl.BlockSpec((1, 64), lambda i: (0, 0)),
            pl.BlockSpec((64, DH), lambda i: (0, 0)),
            pl.BlockSpec((1, DH), lambda i: (0, 0)),
            wspec, aspec, wspec, aspec, wspec, aspec,
        ],
        out_specs=[
            pl.BlockSpec((B, DH), lambda i: (i, 0)),
            pl.BlockSpec((B, 8), lambda i: (i, 0)),
        ],
        out_shape=[
            jax.ShapeDtypeStruct((EE, DH), F32),
            jax.ShapeDtypeStruct((EE, 8), F32),
        ],
    )(x, w1, b1, w2, b2, wes[0], ates[0], wes[1], ates[1], wes[2], ates[2])


def _head(hidden, w, asw, adw, we, ate, seh0, seh1, cnt2):
    h = _dot(hidden, w)
    asrc = _dot(h, asw)
    adst = _dot(h, adw)
    cnt = jnp.maximum(cnt2[:, 0:1] + cnt2[:, 1:2], 1.0)
    smean = (seh0 + seh1) / cnt
    aeloop = _dot(smean, _dot(we, ate))
    al = asrc + adst + aeloop
    al = jnp.maximum(al, 0.2 * al)
    ploop = jnp.exp(al)
    nb = h.shape[0]
    aux = jnp.concatenate([asrc, adst, ploop, jnp.zeros((nb, 5), F32)], axis=1)
    return h, aux


def _combine(acc_a, acc_b, den2, h_prev, ploop_prev, bias_prev):
    num = acc_a + acc_b + ploop_prev * h_prev
    den = den2[:, 0:1] + den2[:, 1:2] + ploop_prev
    return num / (den + 1e-16) + bias_prev


def _stage0_body(nh, w, asw, adw, we, ate, seh0, seh1, cnt2, h_o, aux_o):
    h, aux = _head(nh[...], w[...], asw[...], adw[...], we[...], ate[...],
                   seh0[...], seh1[...], cnt2[...])
    h_o[...] = h
    aux_o[...] = aux


def _stage_body(acc_a, acc_b, den2, h_p, aux_p, bias_p,
                w, asw, adw, we, ate, seh0, seh1, cnt2,
                hid_o, h_o, aux_o):
    hidden = _combine(acc_a[...], acc_b[...], den2[...], h_p[...],
                      aux_p[:, 2:3], bias_p[...])
    hid_o[...] = hidden
    h, aux = _head(hidden, w[...], asw[...], adw[...], we[...], ate[...],
                   seh0[...], seh1[...], cnt2[...])
    h_o[...] = h
    aux_o[...] = aux


_B = 1000
_bspec = {
    'n': pl.BlockSpec((_B, DH), lambda i: (i, 0)),
    'x': pl.BlockSpec((_B, 8), lambda i: (i, 0)),
    'c': pl.BlockSpec((_B, 2), lambda i: (i, 0)),
    'w': pl.BlockSpec((DH, DH), lambda i: (0, 0)),
    'v': pl.BlockSpec((DH, 1), lambda i: (0, 0)),
    'b': pl.BlockSpec((1, DH), lambda i: (0, 0)),
}


def _stage0(nh, w, asw, adw, we, ate, seh0, seh1, cnt2):
    s = _bspec
    return pl.pallas_call(
        _stage0_body,
        grid=(NN // _B,),
        in_specs=[s['n'], s['w'], s['v'], s['v'], s['w'], s['v'],
                  s['n'], s['n'], s['c']],
        out_specs=[s['n'], s['x']],
        out_shape=[jax.ShapeDtypeStruct((NN, DH), F32),
                   jax.ShapeDtypeStruct((NN, 8), F32)],
    )(nh, w, asw, adw, we, ate, seh0, seh1, cnt2)


def _stage(acc_a, acc_b, den2, h_p, aux_p, bias_p, w, asw, adw, we, ate,
           seh0, seh1, cnt2):
    s = _bspec
    return pl.pallas_call(
        _stage_body,
        grid=(NN // _B,),
        in_specs=[s['n'], s['n'], s['c'], s['n'], s['x'], s['b'],
                  s['w'], s['v'], s['v'], s['w'], s['v'],
                  s['n'], s['n'], s['c']],
        out_specs=[s['n'], s['n'], s['x']],
        out_shape=[jax.ShapeDtypeStruct((NN, DH), F32),
                   jax.ShapeDtypeStruct((NN, DH), F32),
                   jax.ShapeDtypeStruct((NN, 8), F32)],
    )(acc_a, acc_b, den2, h_p, aux_p, bias_p, w, asw, adw, we, ate,
      seh0, seh1, cnt2)


def _latent_body(acc_a, acc_b, den2, h_p, aux_p, bias_p,
                 nh, h1, h2, wa, wb, wc, wd, b1, w2, b2, z_o):
    h3 = _combine(acc_a[...], acc_b[...], den2[...], h_p[...],
                  aux_p[:, 2:3], bias_p[...])
    t = (_dot(nh[...], wa[...]) + _dot(h1[...], wb[...])
         + _dot(h2[...], wc[...]) + _dot(h3, wd[...]) + b1[...])
    t = jnp.tanh(t)
    z_o[...] = jnp.tanh(_dot(t, w2[...]) + b2[...])


def _latent(acc_a, acc_b, den2, h_p, aux_p, bias_p, nh, h1, h2,
            wa, wb, wc, wd, b1, w2, b2):
    s = _bspec
    return pl.pallas_call(
        _latent_body,
        grid=(NN // _B,),
        in_specs=[s['n'], s['n'], s['c'], s['n'], s['x'], s['b'],
                  s['n'], s['n'], s['n'],
                  s['w'], s['w'], s['w'], s['w'], s['b'], s['w'], s['b']],
        out_specs=s['n'],
        out_shape=jax.ShapeDtypeStruct((NN, DH), F32),
    )(acc_a, acc_b, den2, h_p, aux_p, bias_p, nh, h1, h2,
      wa, wb, wc, wd, b1, w2, b2)


_MESH = plsc.VectorSubcoreMesh(core_axis_name="c", subcore_axis_name="s",
                               num_cores=NC, num_subcores=NS)


PCH = 125
PNCH = (EE // NW) // PCH


def _sc_prep_body(dst_r, eh_r, sehp_r, cntp_r,
                  dst_t, rows2_t, ones_t, cz_t, seh_s, cnt_s,
                  semg0, semg1, sems0, sems1):
    cid = lax.axis_index("c")
    sid = lax.axis_index("s")
    tile = cid * NS + sid
    rows_t = rows2_t.at[0]

    def zb(i, c):
        r = i // 8
        col = (i % 8) * 16
        rows2_t[0, r, pl.ds(col, 16)] = jnp.zeros((16,), F32)
        return c
    lax.fori_loop(0, PCH * 8, zb, 0)

    def zc(i, c):
        cz_t[pl.ds(i * 16, 16)] = jnp.zeros((16,), F32)
        return c
    lax.fori_loop(0, 39, zc, 0)

    def ob(i, c):
        ones_t[pl.ds(i * 16, 16)] = jnp.full((16,), 1.0, F32)
        return c
    lax.fori_loop(0, 8, ob, 0)

    base = sid * ROWS_PT
    for q in range(4):
        pltpu.sync_copy(rows_t, seh_s.at[pl.ds(base + q * PCH, PCH)])
    pltpu.sync_copy(rows_t.at[pl.ds(0, 124)], seh_s.at[pl.ds(base + 500, 124)])
    pltpu.sync_copy(cz_t, cnt_s.at[pl.ds(base, ROWS_PT)])

    @pl.when(sid == NS - 1)
    def _():
        pltpu.sync_copy(rows_t.at[pl.ds(0, 16)], seh_s.at[pl.ds(9984, 16)])
        pltpu.sync_copy(cz_t.at[pl.ds(0, 16)], cnt_s.at[pl.ds(9984, 16)])

    pltpu.sync_copy(dst_r.at[pl.ds(tile * PNCH, PNCH)], dst_t)
    plsc.subcore_barrier()

    ebase = tile * (PCH * PNCH)

    def g_issue(j, b, sem):
        pltpu.async_copy(eh_r.at[pl.ds(ebase + j * PCH, PCH)],
                         rows2_t.at[b], sem)

    def g_wait(j, b, sem):
        pltpu.make_async_copy(eh_r.at[pl.ds(ebase + j * PCH, PCH)],
                              rows2_t.at[b], sem).wait()

    def s_issue(j, b, sem):
        pltpu.async_copy(rows2_t.at[b], seh_s.at[dst_t.at[j]], sem, add=True)

    def s_wait(j, b, sem):
        pltpu.make_async_copy(rows2_t.at[b], seh_s.at[dst_t.at[j]], sem).wait()

    g_issue(0, 0, semg0)
    g_issue(1, 1, semg1)

    def cb(g, c):
        for b, sg, ss in ((0, semg0, sems0), (1, semg1, sems1)):
            j = g + b
            g_wait(j, b, sg)
            s_issue(j, b, ss)
            pltpu.sync_copy(ones_t.at[pl.ds(0, PCH)], cnt_s.at[dst_t.at[j]],
                            add=True)
            s_wait(j, b, ss)

            @pl.when(j + 2 < PNCH)
            def _():
                g_issue(j + 2, b, sg)
        return c
    lax.fori_loop(0, PNCH // 2, lambda i, c: cb(2 * i, c), 0)

    plsc.subcore_barrier()
    pltpu.sync_copy(seh_s.at[pl.ds(base, ROWS_PT)],
                    sehp_r.at[cid, pl.ds(base, ROWS_PT)])
    pltpu.sync_copy(cnt_s.at[pl.ds(base, ROWS_PT)],
                    cntp_r.at[cid, pl.ds(base, ROWS_PT)])

    @pl.when(sid == NS - 1)
    def _():
        pltpu.sync_copy(seh_s.at[pl.ds(9984, 16)],
                        sehp_r.at[cid, pl.ds(9984, 16)])
        pltpu.sync_copy(cnt_s.at[pl.ds(9984, 16)],
                        cntp_r.at[cid, pl.ds(9984, 16)])


def _sc_prep(dst2d, eh):
    return pl.kernel(
        _sc_prep_body,
        out_type=[jax.ShapeDtypeStruct((NC, NN, DH), F32),
                  jax.ShapeDtypeStruct((NC, NN), F32)],
        mesh=_MESH,
        compiler_params=_SC_PARAMS,
        scratch_types=[
            pltpu.VMEM((PNCH, PCH), jnp.int32),
            pltpu.VMEM((2, PCH, DH), F32),
            pltpu.VMEM((PCH + 3,), F32),
            pltpu.VMEM((ROWS_PT,), F32),
            pltpu.VMEM_SHARED((NN, DH), F32),
            pltpu.VMEM_SHARED((NN,), F32),
            pltpu.SemaphoreType.DMA,
            pltpu.SemaphoreType.DMA,
            pltpu.SemaphoreType.DMA,
            pltpu.SemaphoreType.DMA,
        ],
    )(dst2d, eh)


def _sc_pass_body(pk_r, ae_r, asrc_r, adst_r, h_r, accp_r, denp_r,
                  pk_t, sidx_t, didx_t, p_t, cz_t, rows3_t,
                  asv_t, adv_t, aev_t, acc_s, den_s,
                  semi0, semi1, semi2, sems0, sems1, sems2):
    cid = lax.axis_index("c")
    sid = lax.axis_index("s")
    tile = cid * NS + sid
    rows_t = rows3_t.at[0]
    semi = (semi0, semi1, semi2)
    sems = (sems0, sems1, sems2)

    def zb(i, c):
        r = i // 8
        col = (i % 8) * 16
        rows3_t[0, r, pl.ds(col, 16)] = jnp.zeros((16,), F32)
        return c
    lax.fori_loop(0, CHUNK * 8, zb, 0)

    def zc(i, c):
        cz_t[pl.ds(i * 16, 16)] = jnp.zeros((16,), F32)
        return c
    lax.fori_loop(0, 39, zc, 0)

    base = sid * ROWS_PT
    for q in range(7):
        pltpu.sync_copy(rows_t, acc_s.at[pl.ds(base + q * CHUNK, CHUNK)])
    pltpu.sync_copy(rows_t.at[pl.ds(0, 64)], acc_s.at[pl.ds(base + 560, 64)])
    pltpu.sync_copy(cz_t, den_s.at[pl.ds(base, ROWS_PT)])

    @pl.when(sid == NS - 1)
    def _():
        pltpu.sync_copy(rows_t.at[pl.ds(0, 16)], acc_s.at[pl.ds(9984, 16)])
        pltpu.sync_copy(cz_t.at[pl.ds(0, 16)], den_s.at[pl.ds(9984, 16)])

    pltpu.sync_copy(pk_r.at[pl.ds(tile * NCHT, NCHT)], pk_t)
    plsc.subcore_barrier()

    def in_issue(j, b):
        for k in range(CHUNK // 16):
            v = pk_t[j, pl.ds(k * 16, 16)]
            sidx_t[b, pl.ds(k * 16, 16)] = jnp.bitwise_and(v, 16383)
            didx_t[b, pl.ds(k * 16, 16)] = lax.shift_right_logical(v, 14)
        pltpu.async_copy(asrc_r.at[sidx_t.at[b]], asv_t.at[b], semi[b])
        pltpu.async_copy(adst_r.at[didx_t.at[b]], adv_t.at[b], semi[b])
        pltpu.async_copy(ae_r.at[tile * NCHT + j], aev_t.at[b], semi[b])
        pltpu.async_copy(h_r.at[sidx_t.at[b]], rows3_t.at[b], semi[b])

    def in_wait(j, b):
        pltpu.make_async_copy(asrc_r.at[sidx_t.at[b]], asv_t.at[b],
                              semi[b]).wait()
        pltpu.make_async_copy(adst_r.at[didx_t.at[b]], adv_t.at[b],
                              semi[b]).wait()
        pltpu.make_async_copy(ae_r.at[tile * NCHT + j], aev_t.at[b],
                              semi[b]).wait()
        pltpu.make_async_copy(h_r.at[sidx_t.at[b]], rows3_t.at[b],
                              semi[b]).wait()

    def sc_wait(b):
        pltpu.make_async_copy(rows3_t.at[b], acc_s.at[didx_t.at[b]],
                              sems[b]).wait()

    in_issue(0, 0)
    in_issue(1, 1)

    def proc(j, b):
        in_wait(j, b)
        for k in range(CHUNK // 16):
            a = (asv_t[b, pl.ds(k * 16, 16)]
                 + adv_t[b, pl.ds(k * 16, 16)]
                 + aev_t[b, pl.ds(k * 16, 16)])
            a = jnp.maximum(a, 0.2 * a)
            p_t[pl.ds(k * 16, 16)] = jnp.exp(a)

        def rb8(i, c2):
            for rr in range(8):
                r = i * 8 + rr
                pv = plsc.load_gather(p_t, [jnp.full((16,), r, jnp.int32)])
                for cc in range(8):
                    rows3_t[b, r, pl.ds(cc * 16, 16)] = (
                        rows3_t[b, r, pl.ds(cc * 16, 16)] * pv)
            return c2
        lax.fori_loop(0, CHUNK // 8, rb8, 0)

        pltpu.sync_copy(p_t, den_s.at[didx_t.at[b]], add=True)
        pltpu.async_copy(rows3_t.at[b], acc_s.at[didx_t.at[b]], sems[b],
                         add=True)
        bp = (b + 2) % 3

        @pl.when(j >= 1)
        def _():
            sc_wait(bp)

        @pl.when(j + 2 < NCHT)
        def _():
            in_issue(j + 2, bp)

    def cb(g, c):
        for b in range(3):
            j = g + b

            @pl.when(j < NCHT)
            def _():
                proc(j, b % 3)
        return c
    lax.fori_loop(0, (NCHT + 2) // 3, lambda i, c: cb(3 * i, c), 0)

    sc_wait((NCHT - 1) % 3)
    plsc.subcore_barrier()
    pltpu.sync_copy(acc_s.at[pl.ds(base, ROWS_PT)],
                    accp_r.at[cid, pl.ds(base, ROWS_PT)])
    pltpu.sync_copy(den_s.at[pl.ds(base, ROWS_PT)],
                    denp_r.at[cid, pl.ds(base, ROWS_PT)])

    @pl.when(sid == NS - 1)
    def _():
        pltpu.sync_copy(acc_s.at[pl.ds(9984, 16)],
                        accp_r.at[cid, pl.ds(9984, 16)])
        pltpu.sync_copy(den_s.at[pl.ds(9984, 16)],
                        denp_r.at[cid, pl.ds(9984, 16)])


def _sc_pass(pk2d, ae2d, asrc, adst, h):
    return pl.kernel(
        _sc_pass_body,
        out_type=[jax.ShapeDtypeStruct((NC, NN, DH), F32),
                  jax.ShapeDtypeStruct((NC, NN), F32)],
        mesh=_MESH,
        compiler_params=_SC_PARAMS,
        scratch_types=[
            pltpu.VMEM((NCHT, CHUNK), jnp.int32),
            pltpu.VMEM((3, CHUNK), jnp.int32),
            pltpu.VMEM((3, CHUNK), jnp.int32),
            pltpu.VMEM((CHUNK,), F32),
            pltpu.VMEM((ROWS_PT,), F32),
            pltpu.VMEM((3, CHUNK, DH), F32),
            pltpu.VMEM((3, CHUNK), F32),
            pltpu.VMEM((3, CHUNK), F32),
            pltpu.VMEM((3, CHUNK), F32),
            pltpu.VMEM_SHARED((NN, DH), F32),
            pltpu.VMEM_SHARED((NN,), F32),
            pltpu.SemaphoreType.DMA,
            pltpu.SemaphoreType.DMA,
            pltpu.SemaphoreType.DMA,
            pltpu.SemaphoreType.DMA,
            pltpu.SemaphoreType.DMA,
            pltpu.SemaphoreType.DMA,
        ],
    )(pk2d, ae2d, asrc, adst, h)


def kernel(adj, node_feat, edge_feat, params):
    p = params
    pk2d = (adj[0] | (adj[1] << 14)).reshape(EE // CHUNK, CHUNK)
    dst2dp = adj[1].reshape(EE // PCH, PCH)

    nh = _node_mlp(node_feat,
                   p['node_fc1_W'], p['node_fc1_b'].reshape(1, 64),
                   p['node_fc2_W'], p['node_fc2_b'].reshape(1, DH))

    wes = [p['gat%d_W_edge' % l] for l in range(3)]
    ates = [p['gat%d_att_edge' % l].reshape(DH, 1) for l in range(3)]
    eh, aex = _edge_mlp(edge_feat,
                        p['edge_fc1_W'], p['edge_fc1_b'].reshape(1, 64),
                        p['edge_fc2_W'], p['edge_fc2_b'].reshape(1, DH),
                        wes, ates)

    sehp, cntp = _sc_prep(dst2dp, eh)
    seh0, seh1 = sehp[0], sehp[1]
    cnt2 = cntp.T

    def layer_w(l):
        return (p['gat%d_W' % l],
                p['gat%d_att_src' % l].reshape(DH, 1),
                p['gat%d_att_dst' % l].reshape(DH, 1),
                p['gat%d_W_edge' % l],
                p['gat%d_att_edge' % l].reshape(DH, 1))

    hcur, aux = _stage0(nh, *layer_w(0), seh0, seh1, cnt2)
    hs = [nh]
    z = None
    for l in range(3):
        ae2d = aex[:, l].reshape(EE // CHUNK, CHUNK)
        accp, denp = _sc_pass(pk2d, ae2d, aux[:, 0], aux[:, 1], hcur)
        den2 = denp.T
        bias_p = p['gat%d_bias' % l].reshape(1, DH)
        if l < 2:
            hidden, hcur, aux = _stage(accp[0], accp[1], den2, hcur, aux,
                                       bias_p, *layer_w(l + 1),
                                       seh0, seh1, cnt2)
            hs.append(hidden)
        else:
            lw = p['latent_fc1_W']
            z = _latent(accp[0], accp[1], den2, hcur, aux, bias_p,
                        hs[0], hs[1], hs[2],
                        lw[0:DH], lw[DH:2 * DH], lw[2 * DH:3 * DH],
                        lw[3 * DH:4 * DH],
                        p['latent_fc1_b'].reshape(1, DH),
                        p['latent_fc2_W'],
                        p['latent_fc2_b'].reshape(1, DH))
    return z, eh


# speedup vs baseline: 28.5938x; 1.0108x over previous
import functools

import jax
import jax.numpy as jnp
from jax import lax
from jax.experimental import pallas as pl
from jax.experimental.pallas import tpu as pltpu
from jax.experimental.pallas import tpu_sc as plsc

NN = 10000
EE = 320000
DH = 128
NC = 2
NS = 16
NW = NC * NS
CHUNK = 80
NCHT = (EE // NW) // CHUNK
ROWS_PT = 624
F32 = jnp.float32

_SC_PARAMS = pltpu.CompilerParams(use_tc_tiling_on_sc=False,
                                  needs_layout_passes=False)


def _dot(a, b):
    return jnp.dot(a, b, preferred_element_type=F32)


def _node_mlp_body(x, w1, b1, w2, b2, o):
    h = jnp.tanh(_dot(x[...], w1[...]) + b1[...])
    o[...] = jnp.tanh(_dot(h, w2[...]) + b2[...])


def _node_mlp(x, w1, b1, w2, b2):
    B = 1000
    return pl.pallas_call(
        _node_mlp_body,
        grid=(NN // B,),
        in_specs=[
            pl.BlockSpec((B, DH), lambda i: (i, 0)),
            pl.BlockSpec((DH, 64), lambda i: (0, 0)),
            pl.BlockSpec((1, 64), lambda i: (0, 0)),
            pl.BlockSpec((64, DH), lambda i: (0, 0)),
            pl.BlockSpec((1, DH), lambda i: (0, 0)),
        ],
        out_specs=pl.BlockSpec((B, DH), lambda i: (i, 0)),
        out_shape=jax.ShapeDtypeStruct((NN, DH), F32),
    )(x, w1, b1, w2, b2)


def _edge_mlp_body(x, w1, b1, w2, b2, we0, ae0, we1, ae1, we2, ae2, eh_o, ax_o):
    h = jnp.tanh(_dot(x[...], w1[...]) + b1[...])
    eh = jnp.tanh(_dot(h, w2[...]) + b2[...])
    eh_o[...] = eh
    c0 = _dot(eh, _dot(we0[...], ae0[...]))
    c1 = _dot(eh, _dot(we1[...], ae1[...]))
    c2 = _dot(eh, _dot(we2[...], ae2[...]))
    z = jnp.zeros((eh.shape[0], 5), F32)
    ax_o[...] = jnp.concatenate([c0, c1, c2, z], axis=1)


def _edge_mlp(x, w1, b1, w2, b2, wes, ates):
    B = 4000
    wspec = pl.BlockSpec((DH, DH), lambda i: (0, 0))
    aspec = pl.BlockSpec((DH, 1), lambda i: (0, 0))
    return pl.pallas_call(
        _edge_mlp_body,
        grid=(EE // B,),
        in_specs=[
            pl.BlockSpec((B, 16), lambda i: (i, 0)),
            pl.BlockSpec((16, 64), lambda i: (0, 0)),
            pl.BlockSpec((1, 64), lambda i: (0, 0)),
            pl.BlockSpec((64, DH), lambda i: (0, 0)),
            pl.BlockSpec((1, DH), lambda i: (0, 0)),
            wspec, aspec, wspec, aspec, wspec, aspec,
        ],
        out_specs=[
            pl.BlockSpec((B, DH), lambda i: (i, 0)),
            pl.BlockSpec((B, 8), lambda i: (i, 0)),
        ],
        out_shape=[
            jax.ShapeDtypeStruct((EE, DH), F32),
            jax.ShapeDtypeStruct((EE, 8), F32),
        ],
    )(x, w1, b1, w2, b2, wes[0], ates[0], wes[1], ates[1], wes[2], ates[2])


def _head(hidden, w, asw, adw, we, ate, seh0, seh1, cnt2):
    h = _dot(hidden, w)
    asrc = _dot(h, asw)
    adst = _dot(h, adw)
    cnt = jnp.maximum(cnt2[:, 0:1] + cnt2[:, 1:2], 1.0)
    smean = (seh0 + seh1) / cnt
    aeloop = _dot(smean, _dot(we, ate))
    al = asrc + adst + aeloop
    al = jnp.maximum(al, 0.2 * al)
    ploop = jnp.exp(al)
    nb = h.shape[0]
    aux = jnp.concatenate([asrc, adst, ploop, jnp.zeros((nb, 5), F32)], axis=1)
    return h, aux


def _combine(acc_a, acc_b, den2, h_prev, ploop_prev, bias_prev):
    num = acc_a + acc_b + ploop_prev * h_prev
    den = den2[:, 0:1] + den2[:, 1:2] + ploop_prev
    return num / (den + 1e-16) + bias_prev


def _stage0_body(nh, w, asw, adw, we, ate, seh0, seh1, cnt2, h_o, aux_o):
    h, aux = _head(nh[...], w[...], asw[...], adw[...], we[...], ate[...],
                   seh0[...], seh1[...], cnt2[...])
    h_o[...] = h
    aux_o[...] = aux


def _stage_body(acc_a, acc_b, den2, h_p, aux_p, bias_p,
                w, asw, adw, we, ate, seh0, seh1, cnt2,
                hid_o, h_o, aux_o):
    hidden = _combine(acc_a[...], acc_b[...], den2[...], h_p[...],
                      aux_p[:, 2:3], bias_p[...])
    hid_o[...] = hidden
    h, aux = _head(hidden, w[...], asw[...], adw[...], we[...], ate[...],
                   seh0[...], seh1[...], cnt2[...])
    h_o[...] = h
    aux_o[...] = aux


_B = 1000
_bspec = {
    'n': pl.BlockSpec((_B, DH), lambda i: (i, 0)),
    'x': pl.BlockSpec((_B, 8), lambda i: (i, 0)),
    'c': pl.BlockSpec((_B, 2), lambda i: (i, 0)),
    'w': pl.BlockSpec((DH, DH), lambda i: (0, 0)),
    'v': pl.BlockSpec((DH, 1), lambda i: (0, 0)),
    'b': pl.BlockSpec((1, DH), lambda i: (0, 0)),
}


def _stage0(nh, w, asw, adw, we, ate, seh0, seh1, cnt2):
    s = _bspec
    return pl.pallas_call(
        _stage0_body,
        grid=(NN // _B,),
        in_specs=[s['n'], s['w'], s['v'], s['v'], s['w'], s['v'],
                  s['n'], s['n'], s['c']],
        out_specs=[s['n'], s['x']],
        out_shape=[jax.ShapeDtypeStruct((NN, DH), F32),
                   jax.ShapeDtypeStruct((NN, 8), F32)],
    )(nh, w, asw, adw, we, ate, seh0, seh1, cnt2)


def _stage(acc_a, acc_b, den2, h_p, aux_p, bias_p, w, asw, adw, we, ate,
           seh0, seh1, cnt2):
    s = _bspec
    return pl.pallas_call(
        _stage_body,
        grid=(NN // _B,),
        in_specs=[s['n'], s['n'], s['c'], s['n'], s['x'], s['b'],
                  s['w'], s['v'], s['v'], s['w'], s['v'],
                  s['n'], s['n'], s['c']],
        out_specs=[s['n'], s['n'], s['x']],
        out_shape=[jax.ShapeDtypeStruct((NN, DH), F32),
                   jax.ShapeDtypeStruct((NN, DH), F32),
                   jax.ShapeDtypeStruct((NN, 8), F32)],
    )(acc_a, acc_b, den2, h_p, aux_p, bias_p, w, asw, adw, we, ate,
      seh0, seh1, cnt2)


def _latent_body(acc_a, acc_b, den2, h_p, aux_p, bias_p,
                 nh, h1, h2, wa, wb, wc, wd, b1, w2, b2, z_o):
    h3 = _combine(acc_a[...], acc_b[...], den2[...], h_p[...],
                  aux_p[:, 2:3], bias_p[...])
    t = (_dot(nh[...], wa[...]) + _dot(h1[...], wb[...])
         + _dot(h2[...], wc[...]) + _dot(h3, wd[...]) + b1[...])
    t = jnp.tanh(t)
    z_o[...] = jnp.tanh(_dot(t, w2[...]) + b2[...])


def _latent(acc_a, acc_b, den2, h_p, aux_p, bias_p, nh, h1, h2,
            wa, wb, wc, wd, b1, w2, b2):
    s = _bspec
    return pl.pallas_call(
        _latent_body,
        grid=(NN // _B,),
        in_specs=[s['n'], s['n'], s['c'], s['n'], s['x'], s['b'],
                  s['n'], s['n'], s['n'],
                  s['w'], s['w'], s['w'], s['w'], s['b'], s['w'], s['b']],
        out_specs=s['n'],
        out_shape=jax.ShapeDtypeStruct((NN, DH), F32),
    )(acc_a, acc_b, den2, h_p, aux_p, bias_p, nh, h1, h2,
      wa, wb, wc, wd, b1, w2, b2)


_MESH = plsc.VectorSubcoreMesh(core_axis_name="c", subcore_axis_name="s",
                               num_cores=NC, num_subcores=NS)


PCH = 125
PNCH = (EE // NW) // PCH


def _sc_prep_body(dst_r, eh_r, sehp_r, cntp_r,
                  dst_t, rows2_t, ones_t, cz_t, seh_s, cnt_s,
                  semg0, semg1, sems0, sems1):
    cid = lax.axis_index("c")
    sid = lax.axis_index("s")
    tile = cid * NS + sid
    rows_t = rows2_t.at[0]

    def zb(i, c):
        r = i // 8
        col = (i % 8) * 16
        rows2_t[0, r, pl.ds(col, 16)] = jnp.zeros((16,), F32)
        return c
    lax.fori_loop(0, PCH * 8, zb, 0)

    def zc(i, c):
        cz_t[pl.ds(i * 16, 16)] = jnp.zeros((16,), F32)
        return c
    lax.fori_loop(0, 39, zc, 0)

    def ob(i, c):
        ones_t[pl.ds(i * 16, 16)] = jnp.full((16,), 1.0, F32)
        return c
    lax.fori_loop(0, 8, ob, 0)

    base = sid * ROWS_PT
    for q in range(4):
        pltpu.sync_copy(rows_t, seh_s.at[pl.ds(base + q * PCH, PCH)])
    pltpu.sync_copy(rows_t.at[pl.ds(0, 124)], seh_s.at[pl.ds(base + 500, 124)])
    pltpu.sync_copy(cz_t, cnt_s.at[pl.ds(base, ROWS_PT)])

    @pl.when(sid == NS - 1)
    def _():
        pltpu.sync_copy(rows_t.at[pl.ds(0, 16)], seh_s.at[pl.ds(9984, 16)])
        pltpu.sync_copy(cz_t.at[pl.ds(0, 16)], cnt_s.at[pl.ds(9984, 16)])

    pltpu.sync_copy(dst_r.at[pl.ds(tile * PNCH, PNCH)], dst_t)
    plsc.subcore_barrier()

    ebase = tile * (PCH * PNCH)

    def g_issue(j, b, sem):
        pltpu.async_copy(eh_r.at[pl.ds(ebase + j * PCH, PCH)],
                         rows2_t.at[b], sem)

    def g_wait(j, b, sem):
        pltpu.make_async_copy(eh_r.at[pl.ds(ebase + j * PCH, PCH)],
                              rows2_t.at[b], sem).wait()

    def s_issue(j, b, sem):
        pltpu.async_copy(rows2_t.at[b], seh_s.at[dst_t.at[j]], sem, add=True)

    def s_wait(j, b, sem):
        pltpu.make_async_copy(rows2_t.at[b], seh_s.at[dst_t.at[j]], sem).wait()

    g_issue(0, 0, semg0)
    g_issue(1, 1, semg1)

    def cb(g, c):
        for b, sg, ss in ((0, semg0, sems0), (1, semg1, sems1)):
            j = g + b
            g_wait(j, b, sg)
            s_issue(j, b, ss)
            pltpu.sync_copy(ones_t.at[pl.ds(0, PCH)], cnt_s.at[dst_t.at[j]],
                            add=True)
            s_wait(j, b, ss)

            @pl.when(j + 2 < PNCH)
            def _():
                g_issue(j + 2, b, sg)
        return c
    lax.fori_loop(0, PNCH // 2, lambda i, c: cb(2 * i, c), 0)

    plsc.subcore_barrier()
    pltpu.sync_copy(seh_s.at[pl.ds(base, ROWS_PT)],
                    sehp_r.at[cid, pl.ds(base, ROWS_PT)])
    pltpu.sync_copy(cnt_s.at[pl.ds(base, ROWS_PT)],
                    cntp_r.at[cid, pl.ds(base, ROWS_PT)])

    @pl.when(sid == NS - 1)
    def _():
        pltpu.sync_copy(seh_s.at[pl.ds(9984, 16)],
                        sehp_r.at[cid, pl.ds(9984, 16)])
        pltpu.sync_copy(cnt_s.at[pl.ds(9984, 16)],
                        cntp_r.at[cid, pl.ds(9984, 16)])


def _sc_prep(dst2d, eh):
    return pl.kernel(
        _sc_prep_body,
        out_type=[jax.ShapeDtypeStruct((NC, NN, DH), F32),
                  jax.ShapeDtypeStruct((NC, NN), F32)],
        mesh=_MESH,
        compiler_params=_SC_PARAMS,
        scratch_types=[
            pltpu.VMEM((PNCH, PCH), jnp.int32),
            pltpu.VMEM((2, PCH, DH), F32),
            pltpu.VMEM((PCH + 3,), F32),
            pltpu.VMEM((ROWS_PT,), F32),
            pltpu.VMEM_SHARED((NN, DH), F32),
            pltpu.VMEM_SHARED((NN,), F32),
            pltpu.SemaphoreType.DMA,
            pltpu.SemaphoreType.DMA,
            pltpu.SemaphoreType.DMA,
            pltpu.SemaphoreType.DMA,
        ],
    )(dst2d, eh)


def _sc_pass_body(pk_r, ae_r, asrc_r, adst_r, h_r, accp_r, denp_r,
                  pk_t, sidx_t, didx_t, p3_t, cz_t, rows3_t,
                  asv_t, adv_t, aev_t, acc_s, den_s,
                  semi0, semi1, semi2, sems0, sems1, sems2):
    cid = lax.axis_index("c")
    sid = lax.axis_index("s")
    tile = cid * NS + sid
    rows_t = rows3_t.at[0]
    semi = (semi0, semi1, semi2)
    sems = (sems0, sems1, sems2)

    def zb(i, c):
        r = i // 8
        col = (i % 8) * 16
        rows3_t[0, r, pl.ds(col, 16)] = jnp.zeros((16,), F32)
        return c
    lax.fori_loop(0, CHUNK * 8, zb, 0)

    def zc(i, c):
        cz_t[pl.ds(i * 16, 16)] = jnp.zeros((16,), F32)
        return c
    lax.fori_loop(0, 39, zc, 0)

    base = sid * ROWS_PT
    for q in range(7):
        pltpu.sync_copy(rows_t, acc_s.at[pl.ds(base + q * CHUNK, CHUNK)])
    pltpu.sync_copy(rows_t.at[pl.ds(0, 64)], acc_s.at[pl.ds(base + 560, 64)])
    pltpu.sync_copy(cz_t, den_s.at[pl.ds(base, ROWS_PT)])

    @pl.when(sid == NS - 1)
    def _():
        pltpu.sync_copy(rows_t.at[pl.ds(0, 16)], acc_s.at[pl.ds(9984, 16)])
        pltpu.sync_copy(cz_t.at[pl.ds(0, 16)], den_s.at[pl.ds(9984, 16)])

    pltpu.sync_copy(pk_r.at[pl.ds(tile * NCHT, NCHT)], pk_t)
    plsc.subcore_barrier()

    def in_issue(j, b):
        for k in range(CHUNK // 16):
            v = pk_t[j, pl.ds(k * 16, 16)]
            sidx_t[b, pl.ds(k * 16, 16)] = jnp.bitwise_and(v, 16383)
            didx_t[b, pl.ds(k * 16, 16)] = lax.shift_right_logical(v, 14)
        pltpu.async_copy(asrc_r.at[sidx_t.at[b]], asv_t.at[b], semi[b])
        pltpu.async_copy(adst_r.at[didx_t.at[b]], adv_t.at[b], semi[b])
        pltpu.async_copy(ae_r.at[tile * NCHT + j], aev_t.at[b], semi[b])
        pltpu.async_copy(h_r.at[sidx_t.at[b]], rows3_t.at[b], semi[b])

    def in_wait(j, b):
        pltpu.make_async_copy(asrc_r.at[sidx_t.at[b]], asv_t.at[b],
                              semi[b]).wait()
        pltpu.make_async_copy(adst_r.at[didx_t.at[b]], adv_t.at[b],
                              semi[b]).wait()
        pltpu.make_async_copy(ae_r.at[tile * NCHT + j], aev_t.at[b],
                              semi[b]).wait()
        pltpu.make_async_copy(h_r.at[sidx_t.at[b]], rows3_t.at[b],
                              semi[b]).wait()

    def sc_wait(b):
        pltpu.make_async_copy(rows3_t.at[b], acc_s.at[didx_t.at[b]],
                              sems[b]).wait()
        pltpu.make_async_copy(p3_t.at[b], den_s.at[didx_t.at[b]],
                              sems[b]).wait()

    in_issue(0, 0)
    in_issue(1, 1)

    def proc(j, b):
        in_wait(j, b)
        for k in range(CHUNK // 16):
            a = (asv_t[b, pl.ds(k * 16, 16)]
                 + adv_t[b, pl.ds(k * 16, 16)]
                 + aev_t[b, pl.ds(k * 16, 16)])
            a = jnp.maximum(a, 0.2 * a)
            p3_t[b, pl.ds(k * 16, 16)] = jnp.exp(a)

        def rb8(i, c2):
            for rr in range(8):
                r = i * 8 + rr
                pv = plsc.load_gather(
                    p3_t, [jnp.full((16,), b, jnp.int32),
                           jnp.full((16,), r, jnp.int32)])
                for cc in range(8):
                    rows3_t[b, r, pl.ds(cc * 16, 16)] = (
                        rows3_t[b, r, pl.ds(cc * 16, 16)] * pv)
            return c2
        lax.fori_loop(0, CHUNK // 8, rb8, 0)

        pltpu.async_copy(p3_t.at[b], den_s.at[didx_t.at[b]], sems[b],
                         add=True)
        pltpu.async_copy(rows3_t.at[b], acc_s.at[didx_t.at[b]], sems[b],
                         add=True)
        bp = (b + 2) % 3

        @pl.when(j >= 1)
        def _():
            sc_wait(bp)

        @pl.when(j + 2 < NCHT)
        def _():
            in_issue(j + 2, bp)

    def cb(g, c):
        for b in range(3):
            j = g + b

            @pl.when(j < NCHT)
            def _():
                proc(j, b % 3)
        return c
    lax.fori_loop(0, (NCHT + 2) // 3, lambda i, c: cb(3 * i, c), 0)

    sc_wait((NCHT - 1) % 3)
    plsc.subcore_barrier()
    pltpu.sync_copy(acc_s.at[pl.ds(base, ROWS_PT)],
                    accp_r.at[cid, pl.ds(base, ROWS_PT)])
    pltpu.sync_copy(den_s.at[pl.ds(base, ROWS_PT)],
                    denp_r.at[cid, pl.ds(base, ROWS_PT)])

    @pl.when(sid == NS - 1)
    def _():
        pltpu.sync_copy(acc_s.at[pl.ds(9984, 16)],
                        accp_r.at[cid, pl.ds(9984, 16)])
        pltpu.sync_copy(den_s.at[pl.ds(9984, 16)],
                        denp_r.at[cid, pl.ds(9984, 16)])


def _sc_pass(pk2d, ae2d, asrc, adst, h):
    return pl.kernel(
        _sc_pass_body,
        out_type=[jax.ShapeDtypeStruct((NC, NN, DH), F32),
                  jax.ShapeDtypeStruct((NC, NN), F32)],
        mesh=_MESH,
        compiler_params=_SC_PARAMS,
        scratch_types=[
            pltpu.VMEM((NCHT, CHUNK), jnp.int32),
            pltpu.VMEM((3, CHUNK), jnp.int32),
            pltpu.VMEM((3, CHUNK), jnp.int32),
            pltpu.VMEM((3, CHUNK), F32),
            pltpu.VMEM((ROWS_PT,), F32),
            pltpu.VMEM((3, CHUNK, DH), F32),
            pltpu.VMEM((3, CHUNK), F32),
            pltpu.VMEM((3, CHUNK), F32),
            pltpu.VMEM((3, CHUNK), F32),
            pltpu.VMEM_SHARED((NN, DH), F32),
            pltpu.VMEM_SHARED((NN,), F32),
            pltpu.SemaphoreType.DMA,
            pltpu.SemaphoreType.DMA,
            pltpu.SemaphoreType.DMA,
            pltpu.SemaphoreType.DMA,
            pltpu.SemaphoreType.DMA,
            pltpu.SemaphoreType.DMA,
        ],
    )(pk2d, ae2d, asrc, adst, h)


def kernel(adj, node_feat, edge_feat, params):
    p = params
    pk2d = (adj[0] | (adj[1] << 14)).reshape(EE // CHUNK, CHUNK)
    dst2dp = adj[1].reshape(EE // PCH, PCH)

    nh = _node_mlp(node_feat,
                   p['node_fc1_W'], p['node_fc1_b'].reshape(1, 64),
                   p['node_fc2_W'], p['node_fc2_b'].reshape(1, DH))

    wes = [p['gat%d_W_edge' % l] for l in range(3)]
    ates = [p['gat%d_att_edge' % l].reshape(DH, 1) for l in range(3)]
    eh, aex = _edge_mlp(edge_feat,
                        p['edge_fc1_W'], p['edge_fc1_b'].reshape(1, 64),
                        p['edge_fc2_W'], p['edge_fc2_b'].reshape(1, DH),
                        wes, ates)

    sehp, cntp = _sc_prep(dst2dp, eh)
    seh0, seh1 = sehp[0], sehp[1]
    cnt2 = cntp.T

    def layer_w(l):
        return (p['gat%d_W' % l],
                p['gat%d_att_src' % l].reshape(DH, 1),
                p['gat%d_att_dst' % l].reshape(DH, 1),
                p['gat%d_W_edge' % l],
                p['gat%d_att_edge' % l].reshape(DH, 1))

    hcur, aux = _stage0(nh, *layer_w(0), seh0, seh1, cnt2)
    hs = [nh]
    z = None
    for l in range(3):
        ae2d = aex[:, l].reshape(EE // CHUNK, CHUNK)
        accp, denp = _sc_pass(pk2d, ae2d, aux[:, 0], aux[:, 1], hcur)
        den2 = denp.T
        bias_p = p['gat%d_bias' % l].reshape(1, DH)
        if l < 2:
            hidden, hcur, aux = _stage(accp[0], accp[1], den2, hcur, aux,
                                       bias_p, *layer_w(l + 1),
                                       seh0, seh1, cnt2)
            hs.append(hidden)
        else:
            lw = p['latent_fc1_W']
            z = _latent(accp[0], accp[1], den2, hcur, aux, bias_p,
                        hs[0], hs[1], hs[2],
                        lw[0:DH], lw[DH:2 * DH], lw[2 * DH:3 * DH],
                        lw[3 * DH:4 * DH],
                        p['latent_fc1_b'].reshape(1, DH),
                        p['latent_fc2_W'],
                        p['latent_fc2_b'].reshape(1, DH))
    return z, eh


# speedup vs baseline: 31.7538x; 1.1105x over previous
import functools

import jax
import jax.numpy as jnp
from jax import lax
from jax.experimental import pallas as pl
from jax.experimental.pallas import tpu as pltpu
from jax.experimental.pallas import tpu_sc as plsc

NN = 10000
EE = 320000
DH = 128
NC = 2
NS = 16
NW = NC * NS
CHUNK = 80
NCHT = (EE // NW) // CHUNK
ROWS_PT = 624
F32 = jnp.float32

_SC_PARAMS = pltpu.CompilerParams(use_tc_tiling_on_sc=False,
                                  needs_layout_passes=False)


def _dot(a, b):
    return jnp.dot(a, b, preferred_element_type=F32)


def _node_mlp_body(x, w1, b1, w2, b2, o):
    h = jnp.tanh(_dot(x[...], w1[...]) + b1[...])
    o[...] = jnp.tanh(_dot(h, w2[...]) + b2[...])


def _node_mlp(x, w1, b1, w2, b2):
    B = 1000
    return pl.pallas_call(
        _node_mlp_body,
        grid=(NN // B,),
        in_specs=[
            pl.BlockSpec((B, DH), lambda i: (i, 0)),
            pl.BlockSpec((DH, 64), lambda i: (0, 0)),
            pl.BlockSpec((1, 64), lambda i: (0, 0)),
            pl.BlockSpec((64, DH), lambda i: (0, 0)),
            pl.BlockSpec((1, DH), lambda i: (0, 0)),
        ],
        out_specs=pl.BlockSpec((B, DH), lambda i: (i, 0)),
        out_shape=jax.ShapeDtypeStruct((NN, DH), F32),
    )(x, w1, b1, w2, b2)


def _edge_mlp_body(x, w1, b1, w2, b2, we0, ae0, we1, ae1, we2, ae2, eh_o, ax_o):
    h = jnp.tanh(_dot(x[...], w1[...]) + b1[...])
    eh = jnp.tanh(_dot(h, w2[...]) + b2[...])
    eh_o[...] = eh
    c0 = _dot(eh, _dot(we0[...], ae0[...]))
    c1 = _dot(eh, _dot(we1[...], ae1[...]))
    c2 = _dot(eh, _dot(we2[...], ae2[...]))
    z = jnp.zeros((eh.shape[0], 5), F32)
    ax_o[...] = jnp.concatenate([c0, c1, c2, z], axis=1)


def _edge_mlp(x, w1, b1, w2, b2, wes, ates):
    B = 4000
    wspec = pl.BlockSpec((DH, DH), lambda i: (0, 0))
    aspec = pl.BlockSpec((DH, 1), lambda i: (0, 0))
    return pl.pallas_call(
        _edge_mlp_body,
        grid=(EE // B,),
        in_specs=[
            pl.BlockSpec((B, 16), lambda i: (i, 0)),
            pl.BlockSpec((16, 64), lambda i: (0, 0)),
            pl.BlockSpec((1, 64), lambda i: (0, 0)),
            pl.BlockSpec((64, DH), lambda i: (0, 0)),
            pl.BlockSpec((1, DH), lambda i: (0, 0)),
            wspec, aspec, wspec, aspec, wspec, aspec,
        ],
        out_specs=[
            pl.BlockSpec((B, DH), lambda i: (i, 0)),
            pl.BlockSpec((B, 8), lambda i: (i, 0)),
        ],
        out_shape=[
            jax.ShapeDtypeStruct((EE, DH), F32),
            jax.ShapeDtypeStruct((EE, 8), F32),
        ],
    )(x, w1, b1, w2, b2, wes[0], ates[0], wes[1], ates[1], wes[2], ates[2])


def _head(hidden, w, asw, adw, we, ate, seh0, seh1, cnt2):
    h = _dot(hidden, w)
    asrc = _dot(h, asw)
    adst = _dot(h, adw)
    cnt = jnp.maximum(cnt2[:, 0:1] + cnt2[:, 1:2], 1.0)
    smean = (seh0 + seh1) / cnt
    aeloop = _dot(smean, _dot(we, ate))
    al = asrc + adst + aeloop
    al = jnp.maximum(al, 0.2 * al)
    ploop = jnp.exp(al)
    nb = h.shape[0]
    aux = jnp.concatenate([asrc, adst, ploop, jnp.zeros((nb, 5), F32)], axis=1)
    return h, aux


def _combine(acc_a, acc_b, den2, h_prev, ploop_prev, bias_prev):
    num = acc_a + acc_b + ploop_prev * h_prev
    den = den2[:, 0:1] + den2[:, 1:2] + ploop_prev
    return num / (den + 1e-16) + bias_prev


def _stage0_body(nh, w, asw, adw, we, ate, seh0, seh1, cnt2, h_o, aux_o):
    h, aux = _head(nh[...], w[...], asw[...], adw[...], we[...], ate[...],
                   seh0[...], seh1[...], cnt2[...])
    h_o[...] = h
    aux_o[...] = aux


def _stage_body(acc_a, acc_b, den2, h_p, aux_p, bias_p,
                w, asw, adw, we, ate, seh0, seh1, cnt2,
                hid_o, h_o, aux_o):
    hidden = _combine(acc_a[...], acc_b[...], den2[...], h_p[...],
                      aux_p[:, 2:3], bias_p[...])
    hid_o[...] = hidden
    h, aux = _head(hidden, w[...], asw[...], adw[...], we[...], ate[...],
                   seh0[...], seh1[...], cnt2[...])
    h_o[...] = h
    aux_o[...] = aux


_B = 1000
_bspec = {
    'n': pl.BlockSpec((_B, DH), lambda i: (i, 0)),
    'x': pl.BlockSpec((_B, 8), lambda i: (i, 0)),
    'c': pl.BlockSpec((_B, 2), lambda i: (i, 0)),
    'w': pl.BlockSpec((DH, DH), lambda i: (0, 0)),
    'v': pl.BlockSpec((DH, 1), lambda i: (0, 0)),
    'b': pl.BlockSpec((1, DH), lambda i: (0, 0)),
}


def _stage0(nh, w, asw, adw, we, ate, seh0, seh1, cnt2):
    s = _bspec
    return pl.pallas_call(
        _stage0_body,
        grid=(NN // _B,),
        in_specs=[s['n'], s['w'], s['v'], s['v'], s['w'], s['v'],
                  s['n'], s['n'], s['c']],
        out_specs=[s['n'], s['x']],
        out_shape=[jax.ShapeDtypeStruct((NN, DH), F32),
                   jax.ShapeDtypeStruct((NN, 8), F32)],
    )(nh, w, asw, adw, we, ate, seh0, seh1, cnt2)


def _stage(acc_a, acc_b, den2, h_p, aux_p, bias_p, w, asw, adw, we, ate,
           seh0, seh1, cnt2):
    s = _bspec
    return pl.pallas_call(
        _stage_body,
        grid=(NN // _B,),
        in_specs=[s['n'], s['n'], s['c'], s['n'], s['x'], s['b'],
                  s['w'], s['v'], s['v'], s['w'], s['v'],
                  s['n'], s['n'], s['c']],
        out_specs=[s['n'], s['n'], s['x']],
        out_shape=[jax.ShapeDtypeStruct((NN, DH), F32),
                   jax.ShapeDtypeStruct((NN, DH), F32),
                   jax.ShapeDtypeStruct((NN, 8), F32)],
    )(acc_a, acc_b, den2, h_p, aux_p, bias_p, w, asw, adw, we, ate,
      seh0, seh1, cnt2)


def _latent_body(acc_a, acc_b, den2, h_p, aux_p, bias_p,
                 nh, h1, h2, wa, wb, wc, wd, b1, w2, b2, z_o):
    h3 = _combine(acc_a[...], acc_b[...], den2[...], h_p[...],
                  aux_p[:, 2:3], bias_p[...])
    t = (_dot(nh[...], wa[...]) + _dot(h1[...], wb[...])
         + _dot(h2[...], wc[...]) + _dot(h3, wd[...]) + b1[...])
    t = jnp.tanh(t)
    z_o[...] = jnp.tanh(_dot(t, w2[...]) + b2[...])


def _latent(acc_a, acc_b, den2, h_p, aux_p, bias_p, nh, h1, h2,
            wa, wb, wc, wd, b1, w2, b2):
    s = _bspec
    return pl.pallas_call(
        _latent_body,
        grid=(NN // _B,),
        in_specs=[s['n'], s['n'], s['c'], s['n'], s['x'], s['b'],
                  s['n'], s['n'], s['n'],
                  s['w'], s['w'], s['w'], s['w'], s['b'], s['w'], s['b']],
        out_specs=s['n'],
        out_shape=jax.ShapeDtypeStruct((NN, DH), F32),
    )(acc_a, acc_b, den2, h_p, aux_p, bias_p, nh, h1, h2,
      wa, wb, wc, wd, b1, w2, b2)


_MESH = plsc.VectorSubcoreMesh(core_axis_name="c", subcore_axis_name="s",
                               num_cores=NC, num_subcores=NS)


PCH = 125
PNCH = (EE // NW) // PCH


def _sc_prep_body(dst_r, eh_r, sehp_r, cntp_r,
                  dst_t, rows2_t, ones_t, cz_t, seh_s, cnt_s,
                  semg0, semg1, sems0, sems1):
    cid = lax.axis_index("c")
    sid = lax.axis_index("s")
    tile = cid * NS + sid
    rows_t = rows2_t.at[0]

    def zb(i, c):
        r = i // 8
        col = (i % 8) * 16
        rows2_t[0, r, pl.ds(col, 16)] = jnp.zeros((16,), F32)
        return c
    lax.fori_loop(0, PCH * 8, zb, 0)

    def zc(i, c):
        cz_t[pl.ds(i * 16, 16)] = jnp.zeros((16,), F32)
        return c
    lax.fori_loop(0, 39, zc, 0)

    def ob(i, c):
        ones_t[pl.ds(i * 16, 16)] = jnp.full((16,), 1.0, F32)
        return c
    lax.fori_loop(0, 8, ob, 0)

    base = sid * ROWS_PT
    for q in range(4):
        pltpu.sync_copy(rows_t, seh_s.at[pl.ds(base + q * PCH, PCH)])
    pltpu.sync_copy(rows_t.at[pl.ds(0, 124)], seh_s.at[pl.ds(base + 500, 124)])
    pltpu.sync_copy(cz_t, cnt_s.at[pl.ds(base, ROWS_PT)])

    @pl.when(sid == NS - 1)
    def _():
        pltpu.sync_copy(rows_t.at[pl.ds(0, 16)], seh_s.at[pl.ds(9984, 16)])
        pltpu.sync_copy(cz_t.at[pl.ds(0, 16)], cnt_s.at[pl.ds(9984, 16)])

    pltpu.sync_copy(dst_r.at[pl.ds(tile * PNCH, PNCH)], dst_t)
    plsc.subcore_barrier()

    ebase = tile * (PCH * PNCH)

    def g_issue(j, b, sem):
        pltpu.async_copy(eh_r.at[pl.ds(ebase + j * PCH, PCH)],
                         rows2_t.at[b], sem)

    def g_wait(j, b, sem):
        pltpu.make_async_copy(eh_r.at[pl.ds(ebase + j * PCH, PCH)],
                              rows2_t.at[b], sem).wait()

    def s_issue(j, b, sem):
        pltpu.async_copy(rows2_t.at[b], seh_s.at[dst_t.at[j]], sem, add=True)

    def s_wait(j, b, sem):
        pltpu.make_async_copy(rows2_t.at[b], seh_s.at[dst_t.at[j]], sem).wait()

    g_issue(0, 0, semg0)
    g_issue(1, 1, semg1)

    def cb(g, c):
        for b, sg, ss in ((0, semg0, sems0), (1, semg1, sems1)):
            j = g + b
            g_wait(j, b, sg)
            s_issue(j, b, ss)
            pltpu.sync_copy(ones_t.at[pl.ds(0, PCH)], cnt_s.at[dst_t.at[j]],
                            add=True)
            s_wait(j, b, ss)

            @pl.when(j + 2 < PNCH)
            def _():
                g_issue(j + 2, b, sg)
        return c
    lax.fori_loop(0, PNCH // 2, lambda i, c: cb(2 * i, c), 0)

    plsc.subcore_barrier()
    pltpu.sync_copy(seh_s.at[pl.ds(base, ROWS_PT)],
                    sehp_r.at[cid, pl.ds(base, ROWS_PT)])
    pltpu.sync_copy(cnt_s.at[pl.ds(base, ROWS_PT)],
                    cntp_r.at[cid, pl.ds(base, ROWS_PT)])

    @pl.when(sid == NS - 1)
    def _():
        pltpu.sync_copy(seh_s.at[pl.ds(9984, 16)],
                        sehp_r.at[cid, pl.ds(9984, 16)])
        pltpu.sync_copy(cnt_s.at[pl.ds(9984, 16)],
                        cntp_r.at[cid, pl.ds(9984, 16)])


def _sc_prep(dst2d, eh):
    return pl.kernel(
        _sc_prep_body,
        out_type=[jax.ShapeDtypeStruct((NC, NN, DH), F32),
                  jax.ShapeDtypeStruct((NC, NN), F32)],
        mesh=_MESH,
        compiler_params=_SC_PARAMS,
        scratch_types=[
            pltpu.VMEM((PNCH, PCH), jnp.int32),
            pltpu.VMEM((2, PCH, DH), F32),
            pltpu.VMEM((PCH + 3,), F32),
            pltpu.VMEM((ROWS_PT,), F32),
            pltpu.VMEM_SHARED((NN, DH), F32),
            pltpu.VMEM_SHARED((NN,), F32),
            pltpu.SemaphoreType.DMA,
            pltpu.SemaphoreType.DMA,
            pltpu.SemaphoreType.DMA,
            pltpu.SemaphoreType.DMA,
        ],
    )(dst2d, eh)


def _sc_pass_body(lidx, pk_r, aex_r, aux_r, h_r, accp_r, denp_r,
                  pk_t, sidx_t, didx_t, asidx_t, adidx_t, aeidx_t,
                  p3_t, cz_t, rows3_t,
                  asv_t, adv_t, aev_t, acc_s, den_s,
                  semi0, semi1, semi2, sems0, sems1, sems2):
    cid = lax.axis_index("c")
    sid = lax.axis_index("s")
    tile = cid * NS + sid
    rows_t = rows3_t.at[0]
    semi = (semi0, semi1, semi2)
    sems = (sems0, sems1, sems2)

    def zb(i, c):
        r = i // 8
        col = (i % 8) * 16
        rows3_t[0, r, pl.ds(col, 16)] = jnp.zeros((16,), F32)
        return c
    lax.fori_loop(0, CHUNK * 8, zb, 0)

    def zc(i, c):
        cz_t[pl.ds(i * 16, 16)] = jnp.zeros((16,), F32)
        return c
    lax.fori_loop(0, 39, zc, 0)

    base = sid * ROWS_PT
    for q in range(7):
        pltpu.sync_copy(rows_t, acc_s.at[pl.ds(base + q * CHUNK, CHUNK)])
    pltpu.sync_copy(rows_t.at[pl.ds(0, 64)], acc_s.at[pl.ds(base + 560, 64)])
    pltpu.sync_copy(cz_t, den_s.at[pl.ds(base, ROWS_PT)])

    @pl.when(sid == NS - 1)
    def _():
        pltpu.sync_copy(rows_t.at[pl.ds(0, 16)], acc_s.at[pl.ds(9984, 16)])
        pltpu.sync_copy(cz_t.at[pl.ds(0, 16)], den_s.at[pl.ds(9984, 16)])

    pltpu.sync_copy(pk_r.at[pl.ds(tile * NCHT, NCHT)], pk_t)
    plsc.subcore_barrier()

    ebase = tile * (CHUNK * NCHT)
    iota16 = lax.iota(jnp.int32, 16)

    def in_issue(j, b):
        for k in range(CHUNK // 16):
            v = pk_t[j, pl.ds(k * 16, 16)]
            s = jnp.bitwise_and(v, 16383)
            d = lax.shift_right_logical(v, 14)
            sidx_t[b, pl.ds(k * 16, 16)] = s
            didx_t[b, pl.ds(k * 16, 16)] = d
            asidx_t[b, pl.ds(k * 16, 16)] = lax.shift_left(s, 3)
            adidx_t[b, pl.ds(k * 16, 16)] = lax.shift_left(d, 3) + 1
            eb = ebase + j * CHUNK + k * 16
            aeidx_t[b, pl.ds(k * 16, 16)] = (
                lax.shift_left(eb + iota16, 3) + lidx)
        pltpu.async_copy(aux_r.at[asidx_t.at[b]], asv_t.at[b], semi[b])
        pltpu.async_copy(aux_r.at[adidx_t.at[b]], adv_t.at[b], semi[b])
        pltpu.async_copy(aex_r.at[aeidx_t.at[b]], aev_t.at[b], semi[b])
        pltpu.async_copy(h_r.at[sidx_t.at[b]], rows3_t.at[b], semi[b])

    def in_wait(j, b):
        pltpu.make_async_copy(aux_r.at[asidx_t.at[b]], asv_t.at[b],
                              semi[b]).wait()
        pltpu.make_async_copy(aux_r.at[adidx_t.at[b]], adv_t.at[b],
                              semi[b]).wait()
        pltpu.make_async_copy(aex_r.at[aeidx_t.at[b]], aev_t.at[b],
                              semi[b]).wait()
        pltpu.make_async_copy(h_r.at[sidx_t.at[b]], rows3_t.at[b],
                              semi[b]).wait()

    def sc_wait(b):
        pltpu.make_async_copy(rows3_t.at[b], acc_s.at[didx_t.at[b]],
                              sems[b]).wait()
        pltpu.make_async_copy(p3_t.at[b], den_s.at[didx_t.at[b]],
                              sems[b]).wait()

    in_issue(0, 0)
    in_issue(1, 1)

    def proc(j, b):
        in_wait(j, b)
        for k in range(CHUNK // 16):
            a = (asv_t[b, pl.ds(k * 16, 16)]
                 + adv_t[b, pl.ds(k * 16, 16)]
                 + aev_t[b, pl.ds(k * 16, 16)])
            a = jnp.maximum(a, 0.2 * a)
            p3_t[b, pl.ds(k * 16, 16)] = jnp.exp(a)

        def rb8(i, c2):
            for rr in range(8):
                r = i * 8 + rr
                pv = plsc.load_gather(
                    p3_t, [jnp.full((16,), b, jnp.int32),
                           jnp.full((16,), r, jnp.int32)])
                for cc in range(8):
                    rows3_t[b, r, pl.ds(cc * 16, 16)] = (
                        rows3_t[b, r, pl.ds(cc * 16, 16)] * pv)
            return c2
        lax.fori_loop(0, CHUNK // 8, rb8, 0)

        pltpu.async_copy(p3_t.at[b], den_s.at[didx_t.at[b]], sems[b],
                         add=True)
        pltpu.async_copy(rows3_t.at[b], acc_s.at[didx_t.at[b]], sems[b],
                         add=True)
        bp = (b + 2) % 3

        @pl.when(j >= 1)
        def _():
            sc_wait(bp)

        @pl.when(j + 2 < NCHT)
        def _():
            in_issue(j + 2, bp)

    def cb(g, c):
        for b in range(3):
            j = g + b

            @pl.when(j < NCHT)
            def _():
                proc(j, b % 3)
        return c
    lax.fori_loop(0, (NCHT + 2) // 3, lambda i, c: cb(3 * i, c), 0)

    sc_wait((NCHT - 1) % 3)
    plsc.subcore_barrier()
    pltpu.sync_copy(acc_s.at[pl.ds(base, ROWS_PT)],
                    accp_r.at[cid, pl.ds(base, ROWS_PT)])
    pltpu.sync_copy(den_s.at[pl.ds(base, ROWS_PT)],
                    denp_r.at[cid, pl.ds(base, ROWS_PT)])

    @pl.when(sid == NS - 1)
    def _():
        pltpu.sync_copy(acc_s.at[pl.ds(9984, 16)],
                        accp_r.at[cid, pl.ds(9984, 16)])
        pltpu.sync_copy(den_s.at[pl.ds(9984, 16)],
                        denp_r.at[cid, pl.ds(9984, 16)])


def _sc_pass(pk2d, aexf, auxf, h, lidx):
    return pl.kernel(
        functools.partial(_sc_pass_body, lidx),
        out_type=[jax.ShapeDtypeStruct((NC, NN, DH), F32),
                  jax.ShapeDtypeStruct((NC, NN), F32)],
        mesh=_MESH,
        compiler_params=_SC_PARAMS,
        scratch_types=[
            pltpu.VMEM((NCHT, CHUNK), jnp.int32),
            pltpu.VMEM((3, CHUNK), jnp.int32),
            pltpu.VMEM((3, CHUNK), jnp.int32),
            pltpu.VMEM((3, CHUNK), jnp.int32),
            pltpu.VMEM((3, CHUNK), jnp.int32),
            pltpu.VMEM((3, CHUNK), jnp.int32),
            pltpu.VMEM((3, CHUNK), F32),
            pltpu.VMEM((ROWS_PT,), F32),
            pltpu.VMEM((3, CHUNK, DH), F32),
            pltpu.VMEM((3, CHUNK), F32),
            pltpu.VMEM((3, CHUNK), F32),
            pltpu.VMEM((3, CHUNK), F32),
            pltpu.VMEM_SHARED((NN, DH), F32),
            pltpu.VMEM_SHARED((NN,), F32),
            pltpu.SemaphoreType.DMA,
            pltpu.SemaphoreType.DMA,
            pltpu.SemaphoreType.DMA,
            pltpu.SemaphoreType.DMA,
            pltpu.SemaphoreType.DMA,
            pltpu.SemaphoreType.DMA,
        ],
    )(pk2d, aexf, auxf, h)


def kernel(adj, node_feat, edge_feat, params):
    p = params
    pk2d = (adj[0] | (adj[1] << 14)).reshape(EE // CHUNK, CHUNK)
    dst2dp = adj[1].reshape(EE // PCH, PCH)

    nh = _node_mlp(node_feat,
                   p['node_fc1_W'], p['node_fc1_b'].reshape(1, 64),
                   p['node_fc2_W'], p['node_fc2_b'].reshape(1, DH))

    wes = [p['gat%d_W_edge' % l] for l in range(3)]
    ates = [p['gat%d_att_edge' % l].reshape(DH, 1) for l in range(3)]
    eh, aex = _edge_mlp(edge_feat,
                        p['edge_fc1_W'], p['edge_fc1_b'].reshape(1, 64),
                        p['edge_fc2_W'], p['edge_fc2_b'].reshape(1, DH),
                        wes, ates)

    sehp, cntp = _sc_prep(dst2dp, eh)
    seh0, seh1 = sehp[0], sehp[1]
    cnt2 = cntp.T

    def layer_w(l):
        return (p['gat%d_W' % l],
                p['gat%d_att_src' % l].reshape(DH, 1),
                p['gat%d_att_dst' % l].reshape(DH, 1),
                p['gat%d_W_edge' % l],
                p['gat%d_att_edge' % l].reshape(DH, 1))

    hcur, aux = _stage0(nh, *layer_w(0), seh0, seh1, cnt2)
    hs = [nh]
    z = None
    aexf = aex.reshape(-1)
    for l in range(3):
        accp, denp = _sc_pass(pk2d, aexf, aux.reshape(-1), hcur, l)
        den2 = denp.T
        bias_p = p['gat%d_bias' % l].reshape(1, DH)
        if l < 2:
            hidden, hcur, aux = _stage(accp[0], accp[1], den2, hcur, aux,
                                       bias_p, *layer_w(l + 1),
                                       seh0, seh1, cnt2)
            hs.append(hidden)
        else:
            lw = p['latent_fc1_W']
            z = _latent(accp[0], accp[1], den2, hcur, aux, bias_p,
                        hs[0], hs[1], hs[2],
                        lw[0:DH], lw[DH:2 * DH], lw[2 * DH:3 * DH],
                        lw[3 * DH:4 * DH],
                        p['latent_fc1_b'].reshape(1, DH),
                        p['latent_fc2_W'],
                        p['latent_fc2_b'].reshape(1, DH))
    return z, eh
